# baseline (speedup 1.0000x reference)
import jax
import jax.numpy as jnp
from jax.experimental import pallas as pl

N = 50000
B = 64
EPS = 1e-5


def _gcn(x, W, b, src, dst, norm):
    h = x @ W
    msg = h[src] * norm[:, None]
    agg = jax.ops.segment_sum(msg, dst, num_segments=N)
    return agg + b


def _bn(x, g, be):
    m = jnp.mean(x, axis=0)
    v = jnp.mean((x - m) ** 2, axis=0)
    return (x - m) * jax.lax.rsqrt(v + EPS) * g + be


def _head_kernel(g_ref, F1_ref, fb1_ref, F2_ref, fb2_ref, F3_ref, fb3_ref,
                 logits_ref):
    z = jnp.maximum(jnp.dot(g_ref[...], F1_ref[...],
                            preferred_element_type=jnp.float32) + fb1_ref[...], 0.0)
    z = jnp.maximum(jnp.dot(z, F2_ref[...],
                            preferred_element_type=jnp.float32) + fb2_ref[...], 0.0)
    logits_ref[...] = jnp.dot(z, F3_ref[...],
                              preferred_element_type=jnp.float32) + fb3_ref[...]


def kernel(x, edge_index, batch, W1, b1, g1, be1, W2, b2, g2, be2, W3, b3, g3,
           be3, F1, fb1, F2, fb2, F3, fb3):
    loop = jnp.arange(N, dtype=edge_index.dtype)
    src = jnp.concatenate([edge_index[0], loop])
    dst = jnp.concatenate([edge_index[1], loop])
    deg = jax.ops.segment_sum(jnp.ones_like(dst, dtype=jnp.float32), dst,
                              num_segments=N)
    dinv = jnp.where(deg > 0, jax.lax.rsqrt(jnp.maximum(deg, 1.0)), 0.0)
    norm = dinv[src] * dinv[dst]

    h = _gcn(x, W1, b1, src, dst, norm)
    h = jax.nn.relu(_bn(h, g1, be1))
    h = _gcn(h, W2, b2, src, dst, norm)
    h = jax.nn.relu(_bn(h, g2, be2))
    h = _gcn(h, W3, b3, src, dst, norm)
    h = jax.nn.relu(_bn(h, g3, be3))

    cnt = jax.ops.segment_sum(jnp.ones((N,), jnp.float32), batch,
                              num_segments=B)
    x_mean = jax.ops.segment_sum(h, batch, num_segments=B) / jnp.maximum(
        cnt, 1.0)[:, None]
    x_max = jax.ops.segment_max(h, batch, num_segments=B)
    x_max = jnp.where(cnt[:, None] > 0, x_max, 0.0)
    g_emb = jnp.concatenate([x_mean, x_max], axis=1)

    logits = pl.pallas_call(
        _head_kernel,
        out_shape=jax.ShapeDtypeStruct((B, 2), jnp.float32),
    )(g_emb, F1, fb1, F2, fb2, F3, fb3)
    return (logits, g_emb)


# speedup vs baseline: 5.7688x; 5.7688x over previous
import functools

import jax
import jax.numpy as jnp
from jax import lax
from jax.experimental import pallas as pl
from jax.experimental.pallas import tpu as pltpu
from jax.experimental.pallas import tpu_sc as plsc

N = 50000
E = 800000
IN_DIM = 12
HID = 64
B = 64
EPS = 1e-5

NC = 2
NS = 16
L = 16

NPAD = 51200
HALF = NPAD // 2
PERSUB = HALF // NS
EPAD = 819200
EPS_SC = EPAD // NS
CHUNK_ROWS = 16
NCHUNK = EPS_SC // (CHUNK_ROWS * 128)
EPW = EPAD // (NC * NS)
RB = NPAD // 16

_mesh = plsc.VectorSubcoreMesh(core_axis_name="c", subcore_axis_name="s")


EROWS_W = EPW // 128
HSLICE = NPAD // NS


@functools.partial(
    pl.kernel,
    out_type=jax.ShapeDtypeStruct((NC, NPAD), jnp.float32),
    mesh=_mesh,
    scratch_types=[
        pltpu.VMEM_SHARED((NPAD,), jnp.float32),
        pltpu.VMEM((HSLICE,), jnp.float32),
        pltpu.VMEM((EROWS_W, 128), jnp.int32),
        pltpu.VMEM((128,), jnp.float32),
        pltpu.SemaphoreType.DMA,
    ],
)
def _sc_degree(dst2d_hbm, out_hbm, hist, zbuf, buf, ones, sem):
    c = lax.axis_index("c")
    s = lax.axis_index("s")
    w = s * NC + c

    def zero_body(i, _):
        zbuf[pl.ds(i * L, L)] = jnp.zeros((L,), jnp.float32)
        return 0

    lax.fori_loop(0, HSLICE // L, zero_body, 0)
    for i in range(128 // L):
        ones[pl.ds(i * L, L)] = jnp.ones((L,), jnp.float32)
    pltpu.sync_copy(zbuf, hist.at[pl.ds(s * HSLICE, HSLICE)])
    pltpu.sync_copy(dst2d_hbm.at[pl.ds(w * EROWS_W, EROWS_W), :], buf)
    plsc.subcore_barrier()

    def add_body(jg, _):
        descs = [
            pltpu.async_copy(ones, hist.at[buf.at[jg * 4 + jj]], sem,
                             add=True)
            for jj in range(4)
        ]
        for d in descs:
            d.wait()
        return 0

    lax.fori_loop(0, EROWS_W // 4, add_body, 0)
    plsc.subcore_barrier()
    pltpu.sync_copy(hist.at[pl.ds(s * HSLICE, HSLICE)],
                    out_hbm.at[c, pl.ds(s * HSLICE, HSLICE)])


@functools.partial(
    pl.kernel,
    out_type=jax.ShapeDtypeStruct((NPAD, HID), jnp.float32),
    mesh=_mesh,
    compiler_params=pltpu.CompilerParams(use_tc_tiling_on_sc=False),
    scratch_types=[
        pltpu.VMEM_SHARED((HALF + 8, HID), jnp.float32),
        pltpu.VMEM((256, HID), jnp.float32),
        pltpu.VMEM((CHUNK_ROWS, 128), jnp.int32),
        pltpu.VMEM((CHUNK_ROWS, 128), jnp.int32),
        pltpu.VMEM((CHUNK_ROWS, 128), jnp.int32),
        pltpu.SemaphoreType.DMA,
    ],
)
def _sc_scatter(u_hbm, src2d_hbm, dst2d_hbm, out_hbm, acc, rowbuf, srcbuf,
                dstbuf, dstloc, sem):
    c = lax.axis_index("c")
    s = lax.axis_index("s")
    chalf = c * HALF

    pltpu.sync_copy(u_hbm.at[pl.ds(chalf + s * PERSUB, PERSUB), :],
                    acc.at[pl.ds(s * PERSUB, PERSUB), :])
    plsc.subcore_barrier()

    def chunk_body(b, _):
        row0 = s * (EPS_SC // 128) + b * CHUNK_ROWS
        pltpu.sync_copy(src2d_hbm.at[pl.ds(row0, CHUNK_ROWS), :], srcbuf)
        pltpu.sync_copy(dst2d_hbm.at[pl.ds(row0, CHUNK_ROWS), :], dstbuf)
        for j in range(CHUNK_ROWS):
            def lane_body(k, _):
                d = dstbuf[j, pl.ds(k * L, L)]
                local = d - chalf
                ok = jnp.logical_and(local >= 0, local < HALF)
                dstloc[j, pl.ds(k * L, L)] = jnp.where(ok, local, HALF)
                return 0

            lax.fori_loop(0, 128 // L, lane_body, 0)
        for jj in range(CHUNK_ROWS // 2):
            descs = [
                pltpu.async_copy(u_hbm.at[srcbuf.at[2 * jj + t]],
                                 rowbuf.at[pl.ds(t * 128, 128), :], sem)
                for t in range(2)
            ]
            for d in descs:
                d.wait()
            for t in range(2):
                pltpu.sync_copy(rowbuf.at[pl.ds(t * 128, 128), :],
                                acc.at[dstloc.at[2 * jj + t]], add=True)
        return 0

    lax.fori_loop(0, NCHUNK, chunk_body, 0)
    plsc.subcore_barrier()
    pltpu.sync_copy(acc.at[pl.ds(s * PERSUB, PERSUB), :],
                    out_hbm.at[pl.ds(chalf + s * PERSUB, PERSUB), :])


RB1 = RB


def _k1_body(degp_ref, x_ref, W1_ref, dinv_ref, u1_ref):
    pid = pl.program_id(0)
    deg = jnp.sum(degp_ref[...], axis=0) + 1.0
    rows = jax.lax.broadcasted_iota(jnp.int32, (RB1, 1), 0) + pid * RB1
    dinv = jnp.where(rows < N, jax.lax.rsqrt(deg)[:, None], 0.0)
    dinv_ref[...] = dinv
    h = jnp.dot(x_ref[...], W1_ref[...], preferred_element_type=jnp.float32)
    u1_ref[...] = h * dinv


def _k1(degp, x_pad, W1):
    return pl.pallas_call(
        _k1_body,
        grid=(16,),
        in_specs=[
            pl.BlockSpec((NC, RB1), lambda i: (0, i)),
            pl.BlockSpec((RB1, IN_DIM), lambda i: (i, 0)),
            pl.BlockSpec((IN_DIM, HID), lambda i: (0, 0)),
        ],
        out_specs=[
            pl.BlockSpec((RB1, 1), lambda i: (i, 0)),
            pl.BlockSpec((RB1, HID), lambda i: (i, 0)),
        ],
        out_shape=[
            jax.ShapeDtypeStruct((NPAD, 1), jnp.float32),
            jax.ShapeDtypeStruct((NPAD, HID), jnp.float32),
        ],
    )(degp, x_pad, W1)


def _k2a_body(acc_ref, dinv_ref, b_ref, g_ref, be_ref, W_ref, ua_ref, ub_ref,
              sums, sumsq, mv):
    p = pl.program_id(0)
    blk = pl.program_id(1)
    dinv = dinv_ref[...]
    agg = dinv * acc_ref[...] + b_ref[...]
    rows = jax.lax.broadcasted_iota(jnp.int32, (RB, 1), 0) + blk * RB
    mask = rows < N

    @pl.when(jnp.logical_and(p == 0, blk == 0))
    def _():
        sums[...] = jnp.zeros_like(sums)
        sumsq[...] = jnp.zeros_like(sumsq)

    @pl.when(p == 0)
    def _():
        am = jnp.where(mask, agg, 0.0)
        sums[...] += jnp.sum(am, axis=0, keepdims=True)
        sumsq[...] += jnp.sum(jnp.where(mask, agg * agg, 0.0), axis=0,
                              keepdims=True)

    @pl.when(jnp.logical_and(p == 0, blk == 15))
    def _():
        m = sums[...] / N
        v = sumsq[...] / N - m * m
        mv[0:1, :] = m
        mv[1:2, :] = jax.lax.rsqrt(v + EPS)

    @pl.when(p == 1)
    def _():
        xn = (agg - mv[0:1, :]) * mv[1:2, :] * g_ref[...] + be_ref[...]
        h = jnp.maximum(xn, 0.0)
        u = jnp.dot(h, W_ref[...], preferred_element_type=jnp.float32) * dinv
        ua_ref[...] = u[:, :HID]
        ub_ref[...] = u[:, HID:]


def _k2a(acc1, dinv, b1, g1, be1, W2):
    return pl.pallas_call(
        _k2a_body,
        grid=(2, 16),
        in_specs=[
            pl.BlockSpec((RB, HID), lambda p, b: (b, 0)),
            pl.BlockSpec((RB, 1), lambda p, b: (b, 0)),
            pl.BlockSpec((1, HID), lambda p, b: (0, 0)),
            pl.BlockSpec((1, HID), lambda p, b: (0, 0)),
            pl.BlockSpec((1, HID), lambda p, b: (0, 0)),
            pl.BlockSpec((HID, 2 * HID), lambda p, b: (0, 0)),
        ],
        out_specs=[
            pl.BlockSpec((RB, HID), lambda p, b: (b, 0)),
            pl.BlockSpec((RB, HID), lambda p, b: (b, 0)),
        ],
        out_shape=[
            jax.ShapeDtypeStruct((NPAD, HID), jnp.float32),
            jax.ShapeDtypeStruct((NPAD, HID), jnp.float32),
        ],
        scratch_shapes=[
            pltpu.VMEM((1, HID), jnp.float32),
            pltpu.VMEM((1, HID), jnp.float32),
            pltpu.VMEM((2, HID), jnp.float32),
        ],
    )(acc1, dinv, b1, g1, be1, W2)


def _k2b_body(acca_ref, accb_ref, dinv_ref, b_ref, g_ref, be_ref, W_ref,
              u_ref, sums, sumsq, mv):
    p = pl.program_id(0)
    blk = pl.program_id(1)
    dinv = dinv_ref[...]
    agg = jnp.concatenate([dinv * acca_ref[...], dinv * accb_ref[...]],
                          axis=1) + b_ref[...]
    rows = jax.lax.broadcasted_iota(jnp.int32, (RB, 1), 0) + blk * RB
    mask = rows < N

    @pl.when(jnp.logical_and(p == 0, blk == 0))
    def _():
        sums[...] = jnp.zeros_like(sums)
        sumsq[...] = jnp.zeros_like(sumsq)

    @pl.when(p == 0)
    def _():
        sums[...] += jnp.sum(jnp.where(mask, agg, 0.0), axis=0, keepdims=True)
        sumsq[...] += jnp.sum(jnp.where(mask, agg * agg, 0.0), axis=0,
                              keepdims=True)

    @pl.when(jnp.logical_and(p == 0, blk == 15))
    def _():
        m = sums[...] / N
        v = sumsq[...] / N - m * m
        mv[0:1, :] = m
        mv[1:2, :] = jax.lax.rsqrt(v + EPS)

    @pl.when(p == 1)
    def _():
        xn = (agg - mv[0:1, :]) * mv[1:2, :] * g_ref[...] + be_ref[...]
        h = jnp.maximum(xn, 0.0)
        u_ref[...] = jnp.dot(h, W_ref[...],
                             preferred_element_type=jnp.float32) * dinv


def _k2b(acca, accb, dinv, b2, g2, be2, W3):
    return pl.pallas_call(
        _k2b_body,
        grid=(2, 16),
        in_specs=[
            pl.BlockSpec((RB, HID), lambda p, b: (b, 0)),
            pl.BlockSpec((RB, HID), lambda p, b: (b, 0)),
            pl.BlockSpec((RB, 1), lambda p, b: (b, 0)),
            pl.BlockSpec((1, 2 * HID), lambda p, b: (0, 0)),
            pl.BlockSpec((1, 2 * HID), lambda p, b: (0, 0)),
            pl.BlockSpec((1, 2 * HID), lambda p, b: (0, 0)),
            pl.BlockSpec((2 * HID, HID), lambda p, b: (0, 0)),
        ],
        out_specs=[pl.BlockSpec((RB, HID), lambda p, b: (b, 0))],
        out_shape=[jax.ShapeDtypeStruct((NPAD, HID), jnp.float32)],
        scratch_shapes=[
            pltpu.VMEM((1, 2 * HID), jnp.float32),
            pltpu.VMEM((1, 2 * HID), jnp.float32),
            pltpu.VMEM((2, 2 * HID), jnp.float32),
        ],
    )(acca, accb, dinv, b2, g2, be2, W3)


def _k3_body(acc_ref, dinv_ref, b_ref, g_ref, be_ref, batch_ref, F1_ref,
             fb1_ref, F2_ref, fb2_ref, F3_ref, fb3_ref, logits_ref, gemb_ref,
             sums, sumsq, mv, psum, pmax, pcnt):
    p = pl.program_id(0)
    blk = pl.program_id(1)
    agg = dinv_ref[...] * acc_ref[...] + b_ref[...]
    rows = jax.lax.broadcasted_iota(jnp.int32, (RB, 1), 0) + blk * RB
    mask = rows < N

    @pl.when(jnp.logical_and(p == 0, blk == 0))
    def _():
        sums[...] = jnp.zeros_like(sums)
        sumsq[...] = jnp.zeros_like(sumsq)

    @pl.when(p == 0)
    def _():
        sums[...] += jnp.sum(jnp.where(mask, agg, 0.0), axis=0, keepdims=True)
        sumsq[...] += jnp.sum(jnp.where(mask, agg * agg, 0.0), axis=0,
                              keepdims=True)

    @pl.when(jnp.logical_and(p == 0, blk == 15))
    def _():
        m = sums[...] / N
        v = sumsq[...] / N - m * m
        mv[0:1, :] = m
        mv[1:2, :] = jax.lax.rsqrt(v + EPS)

    @pl.when(jnp.logical_and(p == 1, blk == 0))
    def _():
        psum[...] = jnp.zeros_like(psum)
        pmax[...] = jnp.full_like(pmax, -3e38)
        pcnt[...] = jnp.zeros_like(pcnt)

    @pl.when(p == 1)
    def _():
        xn = (agg - mv[0:1, :]) * mv[1:2, :] * g_ref[...] + be_ref[...]
        h = jnp.maximum(xn, 0.0)
        gid = jax.lax.broadcasted_iota(jnp.int32, (1, B), 1)
        oh = jnp.where(jnp.logical_and(mask, batch_ref[...] == gid), 1.0, 0.0)
        psum[...] += jax.lax.dot_general(
            oh, h, (((0,), (0,)), ((), ())),
            preferred_element_type=jnp.float32)
        pcnt[...] += jax.lax.dot_general(
            oh, jnp.ones((RB, 1), jnp.float32), (((0,), (0,)), ((), ())),
            preferred_element_type=jnp.float32)
        for g in range(B):
            colmask = oh[:, g:g + 1] > 0
            mg = jnp.max(jnp.where(colmask, h, -3e38), axis=0, keepdims=True)
            pmax[g:g + 1, :] = jnp.maximum(pmax[g:g + 1, :], mg)

    @pl.when(jnp.logical_and(p == 2, blk == 0))
    def _():
        cnt = pcnt[...]
        xmean = psum[...] / jnp.maximum(cnt, 1.0)
        xmax = jnp.where(cnt > 0, pmax[...], 0.0)
        gemb = jnp.concatenate([xmean, xmax], axis=1)
        gemb_ref[...] = gemb
        z = jnp.maximum(
            jnp.dot(gemb, F1_ref[...], preferred_element_type=jnp.float32)
            + fb1_ref[...], 0.0)
        z = jnp.maximum(
            jnp.dot(z, F2_ref[...], preferred_element_type=jnp.float32)
            + fb2_ref[...], 0.0)
        logits_ref[...] = jnp.dot(
            z, F3_ref[...], preferred_element_type=jnp.float32) + fb3_ref[...]


def _k3(acc3, dinv, b3, g3, be3, batch2d, F1, fb1, F2, fb2, F3, fb3):
    full = lambda shape: pl.BlockSpec(shape, lambda p, b: tuple(
        0 for _ in shape))
    return pl.pallas_call(
        _k3_body,
        grid=(3, 16),
        in_specs=[
            pl.BlockSpec((RB, HID), lambda p, b: (b, 0)),
            pl.BlockSpec((RB, 1), lambda p, b: (b, 0)),
            full((1, HID)),
            full((1, HID)),
            full((1, HID)),
            pl.BlockSpec((RB, 1), lambda p, b: (b, 0)),
            full((2 * HID, HID)),
            full((1, HID)),
            full((HID, HID // 2)),
            full((1, HID // 2)),
            full((HID // 2, 2)),
            full((1, 2)),
        ],
        out_specs=[
            full((B, 2)),
            full((B, 2 * HID)),
        ],
        out_shape=[
            jax.ShapeDtypeStruct((B, 2), jnp.float32),
            jax.ShapeDtypeStruct((B, 2 * HID), jnp.float32),
        ],
        scratch_shapes=[
            pltpu.VMEM((1, HID), jnp.float32),
            pltpu.VMEM((1, HID), jnp.float32),
            pltpu.VMEM((2, HID), jnp.float32),
            pltpu.VMEM((B, HID), jnp.float32),
            pltpu.VMEM((B, HID), jnp.float32),
            pltpu.VMEM((B, 1), jnp.float32),
        ],
    )(acc3, dinv, b3, g3, be3, batch2d, F1, fb1, F2, fb2, F3, fb3)


def kernel(x, edge_index, batch, W1, b1, g1, be1, W2, b2, g2, be2, W3, b3, g3,
           be3, F1, fb1, F2, fb2, F3, fb3):
    src = edge_index[0]
    dst = edge_index[1]
    pad_e = EPAD - E
    srcp = jnp.concatenate([src, jnp.zeros((pad_e,), jnp.int32)])
    dstp = jnp.concatenate([dst, jnp.full((pad_e,), NPAD - 1, jnp.int32)])
    src2d = srcp.reshape(EPAD // 128, 128)
    dst2d = dstp.reshape(EPAD // 128, 128)
    x_pad = jnp.pad(x, ((0, NPAD - N), (0, 0)))
    batch2d = jnp.pad(batch, (0, NPAD - N),
                      constant_values=B - 1).reshape(NPAD, 1)

    degp = _sc_degree(dst2d)
    dinv, u1 = _k1(degp, x_pad, W1)
    acc1 = _sc_scatter(u1, src2d, dst2d)
    u2a, u2b = _k2a(acc1, dinv, b1.reshape(1, -1), g1.reshape(1, -1),
                    be1.reshape(1, -1), W2)
    acc2a = _sc_scatter(u2a, src2d, dst2d)
    acc2b = _sc_scatter(u2b, src2d, dst2d)
    (u3,) = _k2b(acc2a, acc2b, dinv, b2.reshape(1, -1), g2.reshape(1, -1),
                 be2.reshape(1, -1), W3)
    acc3 = _sc_scatter(u3, src2d, dst2d)
    logits, gemb = _k3(acc3, dinv, b3.reshape(1, -1), g3.reshape(1, -1),
                       be3.reshape(1, -1), batch2d, F1, fb1.reshape(1, -1),
                       F2, fb2.reshape(1, -1), F3, fb3.reshape(1, -1))
    return (logits, gemb)


# speedup vs baseline: 9.8610x; 1.7094x over previous
import functools

import jax
import jax.numpy as jnp
from jax import lax
from jax.experimental import pallas as pl
from jax.experimental.pallas import tpu as pltpu
from jax.experimental.pallas import tpu_sc as plsc

N = 50000
E = 800000
IN_DIM = 12
HID = 64
B = 64
EPS = 1e-5

NC = 2
NS = 16
L = 16

NPAD = 51200
EPAD = 819200
CW = HID // NC
ROWS_SUB = EPAD // NS // 128
CHUNK_ROWS = 25
NCHUNK = ROWS_SUB // CHUNK_ROWS
NSLOT = 3
ZROWS = NPAD // NS
RB = NPAD // 16

_mesh = plsc.VectorSubcoreMesh(core_axis_name="c", subcore_axis_name="s")


EPW = EPAD // (NC * NS)
EROWS_W = EPW // 128
HSLICE = NPAD // NS


@functools.partial(
    pl.kernel,
    out_type=jax.ShapeDtypeStruct((NC, NPAD), jnp.float32),
    mesh=_mesh,
    scratch_types=[
        pltpu.VMEM_SHARED((NPAD,), jnp.float32),
        pltpu.VMEM((HSLICE,), jnp.float32),
        pltpu.VMEM((EROWS_W, 128), jnp.int32),
        pltpu.VMEM((128,), jnp.float32),
        pltpu.SemaphoreType.DMA,
    ],
)
def _sc_degree(dst2d_hbm, out_hbm, hist, zbuf, buf, ones, sem):
    c = lax.axis_index("c")
    s = lax.axis_index("s")
    w = s * NC + c

    def zero_body(i, _):
        zbuf[pl.ds(i * L, L)] = jnp.zeros((L,), jnp.float32)
        return 0

    lax.fori_loop(0, HSLICE // L, zero_body, 0)
    for i in range(128 // L):
        ones[pl.ds(i * L, L)] = jnp.ones((L,), jnp.float32)
    pltpu.sync_copy(zbuf, hist.at[pl.ds(s * HSLICE, HSLICE)])
    pltpu.sync_copy(dst2d_hbm.at[pl.ds(w * EROWS_W, EROWS_W), :], buf)
    plsc.subcore_barrier()

    def add_body(jg, _):
        descs = [
            pltpu.async_copy(ones, hist.at[buf.at[jg * 4 + jj]], sem,
                             add=True)
            for jj in range(4)
        ]
        for d in descs:
            d.wait()
        return 0

    lax.fori_loop(0, EROWS_W // 4, add_body, 0)
    plsc.subcore_barrier()
    pltpu.sync_copy(hist.at[pl.ds(s * HSLICE, HSLICE)],
                    out_hbm.at[c, pl.ds(s * HSLICE, HSLICE)])


@functools.partial(
    pl.kernel,
    out_type=jax.ShapeDtypeStruct((NC, NPAD, CW), jnp.float32),
    mesh=_mesh,
    compiler_params=pltpu.CompilerParams(use_tc_tiling_on_sc=False),
    scratch_types=[
        pltpu.VMEM_SHARED((NPAD, CW), jnp.float32),
        pltpu.VMEM((NSLOT * 128, CW), jnp.float32),
        pltpu.VMEM((CHUNK_ROWS, 128), jnp.int32),
        pltpu.VMEM((CHUNK_ROWS, 128), jnp.int32),
        pltpu.SemaphoreType.DMA,
        pltpu.SemaphoreType.DMA,
        pltpu.SemaphoreType.DMA,
        pltpu.SemaphoreType.DMA,
        pltpu.SemaphoreType.DMA,
        pltpu.SemaphoreType.DMA,
    ],
)
def _sc_scatter(u2_hbm, src2d_hbm, dst2d_hbm, out_hbm, acc, rowbuf, srcbuf,
                dstbuf, g0, g1, g2, a0, a1, a2):
    c = lax.axis_index("c")
    s = lax.axis_index("s")
    gsems = (g0, g1, g2)
    asems = (a0, a1, a2)

    def zrow(i, _):
        for k in range(CW // L):
            rowbuf[i, pl.ds(k * L, L)] = jnp.zeros((L,), jnp.float32)
        return 0

    lax.fori_loop(0, 128, zrow, 0)

    def zinit(i, _):
        pltpu.sync_copy(rowbuf.at[pl.ds(0, 128), :],
                        acc.at[pl.ds(s * ZROWS + i * 128, 128), :])
        return 0

    lax.fori_loop(0, ZROWS // 128, zinit, 0)
    plsc.subcore_barrier()

    def chunk_body(b, _):
        row0 = s * ROWS_SUB + b * CHUNK_ROWS
        pltpu.sync_copy(src2d_hbm.at[pl.ds(row0, CHUNK_ROWS), :], srcbuf)
        pltpu.sync_copy(dst2d_hbm.at[pl.ds(row0, CHUNK_ROWS), :], dstbuf)
        for j in range(CHUNK_ROWS):
            def lane_body(k, _):
                v = srcbuf[j, pl.ds(k * L, L)]
                srcbuf[j, pl.ds(k * L, L)] = v * 2 + c
                return 0

            lax.fori_loop(0, 128 // L, lane_body, 0)

        gd = [None] * CHUNK_ROWS
        ad = [None] * CHUNK_ROWS
        for j in range(NSLOT):
            gd[j] = pltpu.async_copy(
                u2_hbm.at[srcbuf.at[j]],
                rowbuf.at[pl.ds(j * 128, 128), :], gsems[j])
        for j in range(CHUNK_ROWS):
            slot = j % NSLOT
            gd[j].wait()
            ad[j] = pltpu.async_copy(
                rowbuf.at[pl.ds(slot * 128, 128), :],
                acc.at[dstbuf.at[j]], asems[slot], add=True)
            nxt = j + NSLOT
            if nxt < CHUNK_ROWS:
                ad[j].wait()
                gd[nxt] = pltpu.async_copy(
                    u2_hbm.at[srcbuf.at[nxt]],
                    rowbuf.at[pl.ds(slot * 128, 128), :], gsems[slot])
        for j in range(CHUNK_ROWS - NSLOT, CHUNK_ROWS):
            ad[j].wait()
        return 0

    lax.fori_loop(0, NCHUNK, chunk_body, 0)
    plsc.subcore_barrier()
    pltpu.sync_copy(acc.at[pl.ds(s * ZROWS, ZROWS), :],
                    out_hbm.at[c, pl.ds(s * ZROWS, ZROWS), :])


RB1 = RB


def _k1_body(degp_ref, x_ref, W1_ref, dinv_ref, u1_ref):
    pid = pl.program_id(0)
    deg = jnp.sum(degp_ref[...], axis=0) + 1.0
    rows = jax.lax.broadcasted_iota(jnp.int32, (RB1, 1), 0) + pid * RB1
    dinv = jnp.where(rows < N, jax.lax.rsqrt(deg)[:, None], 0.0)
    dinv_ref[...] = dinv
    h = jnp.dot(x_ref[...], W1_ref[...], preferred_element_type=jnp.float32)
    u1_ref[...] = h * dinv


def _k1(degp, x_pad, W1):
    return pl.pallas_call(
        _k1_body,
        grid=(16,),
        in_specs=[
            pl.BlockSpec((NC, RB1), lambda i: (0, i)),
            pl.BlockSpec((RB1, IN_DIM), lambda i: (i, 0)),
            pl.BlockSpec((IN_DIM, HID), lambda i: (0, 0)),
        ],
        out_specs=[
            pl.BlockSpec((RB1, 1), lambda i: (i, 0)),
            pl.BlockSpec((RB1, HID), lambda i: (i, 0)),
        ],
        out_shape=[
            jax.ShapeDtypeStruct((NPAD, 1), jnp.float32),
            jax.ShapeDtypeStruct((NPAD, HID), jnp.float32),
        ],
    )(degp, x_pad, W1)


def _k2a_body(acc_ref, u_ref, dinv_ref, b_ref, g_ref, be_ref, W_ref, ua_ref,
              ub_ref, sums, sumsq, mv):
    p = pl.program_id(0)
    blk = pl.program_id(1)
    dinv = dinv_ref[...]
    neigh = jnp.concatenate([acc_ref[0], acc_ref[1]], axis=1) + u_ref[...]
    agg = dinv * neigh + b_ref[...]
    rows = jax.lax.broadcasted_iota(jnp.int32, (RB, 1), 0) + blk * RB
    mask = rows < N

    @pl.when(jnp.logical_and(p == 0, blk == 0))
    def _():
        sums[...] = jnp.zeros_like(sums)
        sumsq[...] = jnp.zeros_like(sumsq)

    @pl.when(p == 0)
    def _():
        am = jnp.where(mask, agg, 0.0)
        sums[...] += jnp.sum(am, axis=0, keepdims=True)
        sumsq[...] += jnp.sum(jnp.where(mask, agg * agg, 0.0), axis=0,
                              keepdims=True)

    @pl.when(jnp.logical_and(p == 0, blk == 15))
    def _():
        m = sums[...] / N
        v = sumsq[...] / N - m * m
        mv[0:1, :] = m
        mv[1:2, :] = jax.lax.rsqrt(v + EPS)

    @pl.when(p == 1)
    def _():
        xn = (agg - mv[0:1, :]) * mv[1:2, :] * g_ref[...] + be_ref[...]
        h = jnp.maximum(xn, 0.0)
        u = jnp.dot(h, W_ref[...], preferred_element_type=jnp.float32) * dinv
        ua_ref[...] = u[:, :HID]
        ub_ref[...] = u[:, HID:]


def _k2a(acc1, u1, dinv, b1, g1, be1, W2):
    return pl.pallas_call(
        _k2a_body,
        grid=(2, 16),
        in_specs=[
            pl.BlockSpec((NC, RB, CW), lambda p, b: (0, b, 0)),
            pl.BlockSpec((RB, HID), lambda p, b: (b, 0)),
            pl.BlockSpec((RB, 1), lambda p, b: (b, 0)),
            pl.BlockSpec((1, HID), lambda p, b: (0, 0)),
            pl.BlockSpec((1, HID), lambda p, b: (0, 0)),
            pl.BlockSpec((1, HID), lambda p, b: (0, 0)),
            pl.BlockSpec((HID, 2 * HID), lambda p, b: (0, 0)),
        ],
        out_specs=[
            pl.BlockSpec((RB, HID), lambda p, b: (b, 0)),
            pl.BlockSpec((RB, HID), lambda p, b: (b, 0)),
        ],
        out_shape=[
            jax.ShapeDtypeStruct((NPAD, HID), jnp.float32),
            jax.ShapeDtypeStruct((NPAD, HID), jnp.float32),
        ],
        scratch_shapes=[
            pltpu.VMEM((1, HID), jnp.float32),
            pltpu.VMEM((1, HID), jnp.float32),
            pltpu.VMEM((2, HID), jnp.float32),
        ],
    )(acc1, u1, dinv, b1, g1, be1, W2)


def _k2b_body(acca_ref, accb_ref, ua_ref, ub_ref, dinv_ref, b_ref, g_ref,
              be_ref, W_ref, u_ref, sums, sumsq, mv):
    p = pl.program_id(0)
    blk = pl.program_id(1)
    dinv = dinv_ref[...]
    neigh = jnp.concatenate(
        [acca_ref[0], acca_ref[1], accb_ref[0], accb_ref[1]], axis=1)
    selfu = jnp.concatenate([ua_ref[...], ub_ref[...]], axis=1)
    agg = dinv * (neigh + selfu) + b_ref[...]
    rows = jax.lax.broadcasted_iota(jnp.int32, (RB, 1), 0) + blk * RB
    mask = rows < N

    @pl.when(jnp.logical_and(p == 0, blk == 0))
    def _():
        sums[...] = jnp.zeros_like(sums)
        sumsq[...] = jnp.zeros_like(sumsq)

    @pl.when(p == 0)
    def _():
        sums[...] += jnp.sum(jnp.where(mask, agg, 0.0), axis=0, keepdims=True)
        sumsq[...] += jnp.sum(jnp.where(mask, agg * agg, 0.0), axis=0,
                              keepdims=True)

    @pl.when(jnp.logical_and(p == 0, blk == 15))
    def _():
        m = sums[...] / N
        v = sumsq[...] / N - m * m
        mv[0:1, :] = m
        mv[1:2, :] = jax.lax.rsqrt(v + EPS)

    @pl.when(p == 1)
    def _():
        xn = (agg - mv[0:1, :]) * mv[1:2, :] * g_ref[...] + be_ref[...]
        h = jnp.maximum(xn, 0.0)
        u_ref[...] = jnp.dot(h, W_ref[...],
                             preferred_element_type=jnp.float32) * dinv


def _k2b(acca, accb, ua, ub, dinv, b2, g2, be2, W3):
    return pl.pallas_call(
        _k2b_body,
        grid=(2, 16),
        in_specs=[
            pl.BlockSpec((NC, RB, CW), lambda p, b: (0, b, 0)),
            pl.BlockSpec((NC, RB, CW), lambda p, b: (0, b, 0)),
            pl.BlockSpec((RB, HID), lambda p, b: (b, 0)),
            pl.BlockSpec((RB, HID), lambda p, b: (b, 0)),
            pl.BlockSpec((RB, 1), lambda p, b: (b, 0)),
            pl.BlockSpec((1, 2 * HID), lambda p, b: (0, 0)),
            pl.BlockSpec((1, 2 * HID), lambda p, b: (0, 0)),
            pl.BlockSpec((1, 2 * HID), lambda p, b: (0, 0)),
            pl.BlockSpec((2 * HID, HID), lambda p, b: (0, 0)),
        ],
        out_specs=[pl.BlockSpec((RB, HID), lambda p, b: (b, 0))],
        out_shape=[jax.ShapeDtypeStruct((NPAD, HID), jnp.float32)],
        scratch_shapes=[
            pltpu.VMEM((1, 2 * HID), jnp.float32),
            pltpu.VMEM((1, 2 * HID), jnp.float32),
            pltpu.VMEM((2, 2 * HID), jnp.float32),
        ],
    )(acca, accb, ua, ub, dinv, b2, g2, be2, W3)


def _k3_body(acc_ref, u_ref, dinv_ref, b_ref, g_ref, be_ref, batch_ref,
             F1_ref, fb1_ref, F2_ref, fb2_ref, F3_ref, fb3_ref, logits_ref,
             gemb_ref, sums, sumsq, mv, psum, pmax, pcnt):
    p = pl.program_id(0)
    blk = pl.program_id(1)
    neigh = jnp.concatenate([acc_ref[0], acc_ref[1]], axis=1) + u_ref[...]
    agg = dinv_ref[...] * neigh + b_ref[...]
    rows = jax.lax.broadcasted_iota(jnp.int32, (RB, 1), 0) + blk * RB
    mask = rows < N

    @pl.when(jnp.logical_and(p == 0, blk == 0))
    def _():
        sums[...] = jnp.zeros_like(sums)
        sumsq[...] = jnp.zeros_like(sumsq)

    @pl.when(p == 0)
    def _():
        sums[...] += jnp.sum(jnp.where(mask, agg, 0.0), axis=0, keepdims=True)
        sumsq[...] += jnp.sum(jnp.where(mask, agg * agg, 0.0), axis=0,
                              keepdims=True)

    @pl.when(jnp.logical_and(p == 0, blk == 15))
    def _():
        m = sums[...] / N
        v = sumsq[...] / N - m * m
        mv[0:1, :] = m
        mv[1:2, :] = jax.lax.rsqrt(v + EPS)

    @pl.when(jnp.logical_and(p == 1, blk == 0))
    def _():
        psum[...] = jnp.zeros_like(psum)
        pmax[...] = jnp.full_like(pmax, -3e38)
        pcnt[...] = jnp.zeros_like(pcnt)

    @pl.when(p == 1)
    def _():
        xn = (agg - mv[0:1, :]) * mv[1:2, :] * g_ref[...] + be_ref[...]
        h = jnp.maximum(xn, 0.0)
        gid = jax.lax.broadcasted_iota(jnp.int32, (1, B), 1)
        oh = jnp.where(jnp.logical_and(mask, batch_ref[...] == gid), 1.0, 0.0)
        psum[...] += jax.lax.dot_general(
            oh, h, (((0,), (0,)), ((), ())),
            preferred_element_type=jnp.float32)
        pcnt[...] += jax.lax.dot_general(
            oh, jnp.ones((RB, 1), jnp.float32), (((0,), (0,)), ((), ())),
            preferred_element_type=jnp.float32)
        for g in range(B):
            colmask = oh[:, g:g + 1] > 0
            mg = jnp.max(jnp.where(colmask, h, -3e38), axis=0, keepdims=True)
            pmax[g:g + 1, :] = jnp.maximum(pmax[g:g + 1, :], mg)

    @pl.when(jnp.logical_and(p == 2, blk == 0))
    def _():
        cnt = pcnt[...]
        xmean = psum[...] / jnp.maximum(cnt, 1.0)
        xmax = jnp.where(cnt > 0, pmax[...], 0.0)
        gemb = jnp.concatenate([xmean, xmax], axis=1)
        gemb_ref[...] = gemb
        z = jnp.maximum(
            jnp.dot(gemb, F1_ref[...], preferred_element_type=jnp.float32)
            + fb1_ref[...], 0.0)
        z = jnp.maximum(
            jnp.dot(z, F2_ref[...], preferred_element_type=jnp.float32)
            + fb2_ref[...], 0.0)
        logits_ref[...] = jnp.dot(
            z, F3_ref[...], preferred_element_type=jnp.float32) + fb3_ref[...]


def _k3(acc3, u3, dinv, b3, g3, be3, batch2d, F1, fb1, F2, fb2, F3, fb3):
    full = lambda shape: pl.BlockSpec(shape, lambda p, b: tuple(
        0 for _ in shape))
    return pl.pallas_call(
        _k3_body,
        grid=(3, 16),
        in_specs=[
            pl.BlockSpec((NC, RB, CW), lambda p, b: (0, b, 0)),
            pl.BlockSpec((RB, HID), lambda p, b: (b, 0)),
            pl.BlockSpec((RB, 1), lambda p, b: (b, 0)),
            full((1, HID)),
            full((1, HID)),
            full((1, HID)),
            pl.BlockSpec((RB, 1), lambda p, b: (b, 0)),
            full((2 * HID, HID)),
            full((1, HID)),
            full((HID, HID // 2)),
            full((1, HID // 2)),
            full((HID // 2, 2)),
            full((1, 2)),
        ],
        out_specs=[
            full((B, 2)),
            full((B, 2 * HID)),
        ],
        out_shape=[
            jax.ShapeDtypeStruct((B, 2), jnp.float32),
            jax.ShapeDtypeStruct((B, 2 * HID), jnp.float32),
        ],
        scratch_shapes=[
            pltpu.VMEM((1, HID), jnp.float32),
            pltpu.VMEM((1, HID), jnp.float32),
            pltpu.VMEM((2, HID), jnp.float32),
            pltpu.VMEM((B, HID), jnp.float32),
            pltpu.VMEM((B, HID), jnp.float32),
            pltpu.VMEM((B, 1), jnp.float32),
        ],
    )(acc3, u3, dinv, b3, g3, be3, batch2d, F1, fb1, F2, fb2, F3, fb3)


def kernel(x, edge_index, batch, W1, b1, g1, be1, W2, b2, g2, be2, W3, b3, g3,
           be3, F1, fb1, F2, fb2, F3, fb3):
    src = edge_index[0]
    dst = edge_index[1]
    pad_e = EPAD - E
    srcp = jnp.concatenate([src, jnp.zeros((pad_e,), jnp.int32)])
    dstp = jnp.concatenate([dst, jnp.full((pad_e,), NPAD - 1, jnp.int32)])
    src2d = srcp.reshape(EPAD // 128, 128)
    dst2d = dstp.reshape(EPAD // 128, 128)
    x_pad = jnp.pad(x, ((0, NPAD - N), (0, 0)))
    batch2d = jnp.pad(batch, (0, NPAD - N),
                      constant_values=B - 1).reshape(NPAD, 1)

    degp = _sc_degree(dst2d)
    dinv, u1 = _k1(degp, x_pad, W1)
    acc1 = _sc_scatter(u1.reshape(NC * NPAD, CW), src2d, dst2d)
    u2a, u2b = _k2a(acc1, u1, dinv, b1.reshape(1, -1), g1.reshape(1, -1),
                    be1.reshape(1, -1), W2)
    acc2a = _sc_scatter(u2a.reshape(NC * NPAD, CW), src2d, dst2d)
    acc2b = _sc_scatter(u2b.reshape(NC * NPAD, CW), src2d, dst2d)
    (u3,) = _k2b(acc2a, acc2b, u2a, u2b, dinv, b2.reshape(1, -1),
                 g2.reshape(1, -1), be2.reshape(1, -1), W3)
    acc3 = _sc_scatter(u3.reshape(NC * NPAD, CW), src2d, dst2d)
    logits, gemb = _k3(acc3, u3, dinv, b3.reshape(1, -1), g3.reshape(1, -1),
                       be3.reshape(1, -1), batch2d, F1, fb1.reshape(1, -1),
                       F2, fb2.reshape(1, -1), F3, fb3.reshape(1, -1))
    return (logits, gemb)


# speedup vs baseline: 10.0263x; 1.0168x over previous
import functools

import jax
import jax.numpy as jnp
from jax import lax
from jax.experimental import pallas as pl
from jax.experimental.pallas import tpu as pltpu
from jax.experimental.pallas import tpu_sc as plsc

N = 50000
E = 800000
IN_DIM = 12
HID = 64
B = 64
EPS = 1e-5

NC = 2
NS = 16
L = 16

NPAD = 51200
EPAD = 819200
CW = HID // NC
ROWS_SUB = EPAD // NS // 128
CHUNK_ROWS = 20
NCHUNK = ROWS_SUB // CHUNK_ROWS
NSLOT = 4
ZROWS = NPAD // NS
RB = NPAD // 16

_mesh = plsc.VectorSubcoreMesh(core_axis_name="c", subcore_axis_name="s")


EPW = EPAD // (NC * NS)
EROWS_W = EPW // 128
HSLICE = NPAD // NS


@functools.partial(
    pl.kernel,
    out_type=jax.ShapeDtypeStruct((NC, NPAD), jnp.float32),
    mesh=_mesh,
    scratch_types=[
        pltpu.VMEM_SHARED((NPAD,), jnp.float32),
        pltpu.VMEM((HSLICE,), jnp.float32),
        pltpu.VMEM((EROWS_W, 128), jnp.int32),
        pltpu.VMEM((128,), jnp.float32),
        pltpu.SemaphoreType.DMA,
    ],
)
def _sc_degree(dst2d_hbm, out_hbm, hist, zbuf, buf, ones, sem):
    c = lax.axis_index("c")
    s = lax.axis_index("s")
    w = s * NC + c

    def zero_body(i, _):
        zbuf[pl.ds(i * L, L)] = jnp.zeros((L,), jnp.float32)
        return 0

    lax.fori_loop(0, HSLICE // L, zero_body, 0)
    for i in range(128 // L):
        ones[pl.ds(i * L, L)] = jnp.ones((L,), jnp.float32)
    pltpu.sync_copy(zbuf, hist.at[pl.ds(s * HSLICE, HSLICE)])
    pltpu.sync_copy(dst2d_hbm.at[pl.ds(w * EROWS_W, EROWS_W), :], buf)
    plsc.subcore_barrier()

    def add_body(jg, _):
        descs = [
            pltpu.async_copy(ones, hist.at[buf.at[jg * 10 + jj]], sem,
                             add=True)
            for jj in range(10)
        ]
        for d in descs:
            d.wait()
        return 0

    lax.fori_loop(0, EROWS_W // 10, add_body, 0)
    plsc.subcore_barrier()
    pltpu.sync_copy(hist.at[pl.ds(s * HSLICE, HSLICE)],
                    out_hbm.at[c, pl.ds(s * HSLICE, HSLICE)])


@functools.partial(
    pl.kernel,
    out_type=jax.ShapeDtypeStruct((NC, NPAD, CW), jnp.float32),
    mesh=_mesh,
    compiler_params=pltpu.CompilerParams(use_tc_tiling_on_sc=False),
    scratch_types=[
        pltpu.VMEM_SHARED((NPAD, CW), jnp.float32),
        pltpu.VMEM((NSLOT * 128, CW), jnp.float32),
        pltpu.VMEM((CHUNK_ROWS, 128), jnp.int32),
        pltpu.VMEM((CHUNK_ROWS, 128), jnp.int32),
        pltpu.SemaphoreType.DMA,
        pltpu.SemaphoreType.DMA,
        pltpu.SemaphoreType.DMA,
        pltpu.SemaphoreType.DMA,
        pltpu.SemaphoreType.DMA,
        pltpu.SemaphoreType.DMA,
        pltpu.SemaphoreType.DMA,
        pltpu.SemaphoreType.DMA,
    ],
)
def _sc_scatter(u2_hbm, src2d_hbm, dst2d_hbm, out_hbm, acc, rowbuf, srcbuf,
                dstbuf, g0, g1, g2, g3, a0, a1, a2, a3):
    c = lax.axis_index("c")
    s = lax.axis_index("s")
    gsems = (g0, g1, g2, g3)
    asems = (a0, a1, a2, a3)

    def zrow(i, _):
        for k in range(CW // L):
            rowbuf[i, pl.ds(k * L, L)] = jnp.zeros((L,), jnp.float32)
        return 0

    lax.fori_loop(0, 128, zrow, 0)

    zd = [
        pltpu.async_copy(rowbuf.at[pl.ds(0, 128), :],
                         acc.at[pl.ds(s * ZROWS + i * 128, 128), :], g0)
        for i in range(ZROWS // 128)
    ]
    for d in zd:
        d.wait()
    plsc.subcore_barrier()

    def chunk_body(b, _):
        row0 = s * ROWS_SUB + b * CHUNK_ROWS
        pltpu.sync_copy(src2d_hbm.at[pl.ds(row0, CHUNK_ROWS), :], srcbuf)
        pltpu.sync_copy(dst2d_hbm.at[pl.ds(row0, CHUNK_ROWS), :], dstbuf)
        for j in range(CHUNK_ROWS):
            for k in range(128 // L):
                v = srcbuf[j, pl.ds(k * L, L)]
                srcbuf[j, pl.ds(k * L, L)] = v * 2 + c

        gd = [None] * CHUNK_ROWS
        ad = [None] * CHUNK_ROWS
        for j in range(NSLOT):
            gd[j] = pltpu.async_copy(
                u2_hbm.at[srcbuf.at[j]],
                rowbuf.at[pl.ds(j * 128, 128), :], gsems[j])
        for j in range(CHUNK_ROWS):
            slot = j % NSLOT
            gd[j].wait()
            ad[j] = pltpu.async_copy(
                rowbuf.at[pl.ds(slot * 128, 128), :],
                acc.at[dstbuf.at[j]], asems[slot], add=True)
            nxt = j + NSLOT
            if nxt < CHUNK_ROWS:
                ad[j].wait()
                gd[nxt] = pltpu.async_copy(
                    u2_hbm.at[srcbuf.at[nxt]],
                    rowbuf.at[pl.ds(slot * 128, 128), :], gsems[slot])
        for j in range(CHUNK_ROWS - NSLOT, CHUNK_ROWS):
            ad[j].wait()
        return 0

    lax.fori_loop(0, NCHUNK, chunk_body, 0)
    plsc.subcore_barrier()
    pltpu.sync_copy(acc.at[pl.ds(s * ZROWS, ZROWS), :],
                    out_hbm.at[c, pl.ds(s * ZROWS, ZROWS), :])


RB1 = RB


def _k1_body(degp_ref, x_ref, W1_ref, dinv_ref, u1_ref):
    pid = pl.program_id(0)
    deg = jnp.sum(degp_ref[...], axis=0) + 1.0
    rows = jax.lax.broadcasted_iota(jnp.int32, (RB1, 1), 0) + pid * RB1
    dinv = jnp.where(rows < N, jax.lax.rsqrt(deg)[:, None], 0.0)
    dinv_ref[...] = dinv
    h = jnp.dot(x_ref[...], W1_ref[...], preferred_element_type=jnp.float32)
    u1_ref[...] = h * dinv


def _k1(degp, x_pad, W1):
    return pl.pallas_call(
        _k1_body,
        grid=(16,),
        in_specs=[
            pl.BlockSpec((NC, RB1), lambda i: (0, i)),
            pl.BlockSpec((RB1, IN_DIM), lambda i: (i, 0)),
            pl.BlockSpec((IN_DIM, HID), lambda i: (0, 0)),
        ],
        out_specs=[
            pl.BlockSpec((RB1, 1), lambda i: (i, 0)),
            pl.BlockSpec((RB1, HID), lambda i: (i, 0)),
        ],
        out_shape=[
            jax.ShapeDtypeStruct((NPAD, 1), jnp.float32),
            jax.ShapeDtypeStruct((NPAD, HID), jnp.float32),
        ],
    )(degp, x_pad, W1)


def _k2a_body(acc_ref, u_ref, dinv_ref, b_ref, g_ref, be_ref, W_ref, ua_ref,
              ub_ref, sums, sumsq, mv):
    p = pl.program_id(0)
    blk = pl.program_id(1)
    dinv = dinv_ref[...]
    neigh = jnp.concatenate([acc_ref[0], acc_ref[1]], axis=1) + u_ref[...]
    agg = dinv * neigh + b_ref[...]
    rows = jax.lax.broadcasted_iota(jnp.int32, (RB, 1), 0) + blk * RB
    mask = rows < N

    @pl.when(jnp.logical_and(p == 0, blk == 0))
    def _():
        sums[...] = jnp.zeros_like(sums)
        sumsq[...] = jnp.zeros_like(sumsq)

    @pl.when(p == 0)
    def _():
        am = jnp.where(mask, agg, 0.0)
        sums[...] += jnp.sum(am, axis=0, keepdims=True)
        sumsq[...] += jnp.sum(jnp.where(mask, agg * agg, 0.0), axis=0,
                              keepdims=True)

    @pl.when(jnp.logical_and(p == 0, blk == 15))
    def _():
        m = sums[...] / N
        v = sumsq[...] / N - m * m
        mv[0:1, :] = m
        mv[1:2, :] = jax.lax.rsqrt(v + EPS)

    @pl.when(p == 1)
    def _():
        xn = (agg - mv[0:1, :]) * mv[1:2, :] * g_ref[...] + be_ref[...]
        h = jnp.maximum(xn, 0.0)
        u = jnp.dot(h, W_ref[...], preferred_element_type=jnp.float32) * dinv
        ua_ref[...] = u[:, :HID]
        ub_ref[...] = u[:, HID:]


def _k2a(acc1, u1, dinv, b1, g1, be1, W2):
    return pl.pallas_call(
        _k2a_body,
        grid=(2, 16),
        in_specs=[
            pl.BlockSpec((NC, RB, CW), lambda p, b: (0, b, 0)),
            pl.BlockSpec((RB, HID), lambda p, b: (b, 0)),
            pl.BlockSpec((RB, 1), lambda p, b: (b, 0)),
            pl.BlockSpec((1, HID), lambda p, b: (0, 0)),
            pl.BlockSpec((1, HID), lambda p, b: (0, 0)),
            pl.BlockSpec((1, HID), lambda p, b: (0, 0)),
            pl.BlockSpec((HID, 2 * HID), lambda p, b: (0, 0)),
        ],
        out_specs=[
            pl.BlockSpec((RB, HID), lambda p, b: (b, 0)),
            pl.BlockSpec((RB, HID), lambda p, b: (b, 0)),
        ],
        out_shape=[
            jax.ShapeDtypeStruct((NPAD, HID), jnp.float32),
            jax.ShapeDtypeStruct((NPAD, HID), jnp.float32),
        ],
        scratch_shapes=[
            pltpu.VMEM((1, HID), jnp.float32),
            pltpu.VMEM((1, HID), jnp.float32),
            pltpu.VMEM((2, HID), jnp.float32),
        ],
    )(acc1, u1, dinv, b1, g1, be1, W2)


def _k2b_body(acca_ref, accb_ref, ua_ref, ub_ref, dinv_ref, b_ref, g_ref,
              be_ref, W_ref, u_ref, sums, sumsq, mv):
    p = pl.program_id(0)
    blk = pl.program_id(1)
    dinv = dinv_ref[...]
    neigh = jnp.concatenate(
        [acca_ref[0], acca_ref[1], accb_ref[0], accb_ref[1]], axis=1)
    selfu = jnp.concatenate([ua_ref[...], ub_ref[...]], axis=1)
    agg = dinv * (neigh + selfu) + b_ref[...]
    rows = jax.lax.broadcasted_iota(jnp.int32, (RB, 1), 0) + blk * RB
    mask = rows < N

    @pl.when(jnp.logical_and(p == 0, blk == 0))
    def _():
        sums[...] = jnp.zeros_like(sums)
        sumsq[...] = jnp.zeros_like(sumsq)

    @pl.when(p == 0)
    def _():
        sums[...] += jnp.sum(jnp.where(mask, agg, 0.0), axis=0, keepdims=True)
        sumsq[...] += jnp.sum(jnp.where(mask, agg * agg, 0.0), axis=0,
                              keepdims=True)

    @pl.when(jnp.logical_and(p == 0, blk == 15))
    def _():
        m = sums[...] / N
        v = sumsq[...] / N - m * m
        mv[0:1, :] = m
        mv[1:2, :] = jax.lax.rsqrt(v + EPS)

    @pl.when(p == 1)
    def _():
        xn = (agg - mv[0:1, :]) * mv[1:2, :] * g_ref[...] + be_ref[...]
        h = jnp.maximum(xn, 0.0)
        u_ref[...] = jnp.dot(h, W_ref[...],
                             preferred_element_type=jnp.float32) * dinv


def _k2b(acca, accb, ua, ub, dinv, b2, g2, be2, W3):
    return pl.pallas_call(
        _k2b_body,
        grid=(2, 16),
        in_specs=[
            pl.BlockSpec((NC, RB, CW), lambda p, b: (0, b, 0)),
            pl.BlockSpec((NC, RB, CW), lambda p, b: (0, b, 0)),
            pl.BlockSpec((RB, HID), lambda p, b: (b, 0)),
            pl.BlockSpec((RB, HID), lambda p, b: (b, 0)),
            pl.BlockSpec((RB, 1), lambda p, b: (b, 0)),
            pl.BlockSpec((1, 2 * HID), lambda p, b: (0, 0)),
            pl.BlockSpec((1, 2 * HID), lambda p, b: (0, 0)),
            pl.BlockSpec((1, 2 * HID), lambda p, b: (0, 0)),
            pl.BlockSpec((2 * HID, HID), lambda p, b: (0, 0)),
        ],
        out_specs=[pl.BlockSpec((RB, HID), lambda p, b: (b, 0))],
        out_shape=[jax.ShapeDtypeStruct((NPAD, HID), jnp.float32)],
        scratch_shapes=[
            pltpu.VMEM((1, 2 * HID), jnp.float32),
            pltpu.VMEM((1, 2 * HID), jnp.float32),
            pltpu.VMEM((2, 2 * HID), jnp.float32),
        ],
    )(acca, accb, ua, ub, dinv, b2, g2, be2, W3)


def _k3_body(acc_ref, u_ref, dinv_ref, b_ref, g_ref, be_ref, batch_ref,
             F1_ref, fb1_ref, F2_ref, fb2_ref, F3_ref, fb3_ref, logits_ref,
             gemb_ref, sums, sumsq, mv, psum, pmax, pcnt):
    p = pl.program_id(0)
    blk = pl.program_id(1)
    neigh = jnp.concatenate([acc_ref[0], acc_ref[1]], axis=1) + u_ref[...]
    agg = dinv_ref[...] * neigh + b_ref[...]
    rows = jax.lax.broadcasted_iota(jnp.int32, (RB, 1), 0) + blk * RB
    mask = rows < N

    @pl.when(jnp.logical_and(p == 0, blk == 0))
    def _():
        sums[...] = jnp.zeros_like(sums)
        sumsq[...] = jnp.zeros_like(sumsq)

    @pl.when(p == 0)
    def _():
        sums[...] += jnp.sum(jnp.where(mask, agg, 0.0), axis=0, keepdims=True)
        sumsq[...] += jnp.sum(jnp.where(mask, agg * agg, 0.0), axis=0,
                              keepdims=True)

    @pl.when(jnp.logical_and(p == 0, blk == 15))
    def _():
        m = sums[...] / N
        v = sumsq[...] / N - m * m
        mv[0:1, :] = m
        mv[1:2, :] = jax.lax.rsqrt(v + EPS)

    @pl.when(jnp.logical_and(p == 1, blk == 0))
    def _():
        psum[...] = jnp.zeros_like(psum)
        pmax[...] = jnp.full_like(pmax, -3e38)
        pcnt[...] = jnp.zeros_like(pcnt)

    @pl.when(p == 1)
    def _():
        xn = (agg - mv[0:1, :]) * mv[1:2, :] * g_ref[...] + be_ref[...]
        h = jnp.maximum(xn, 0.0)
        gid = jax.lax.broadcasted_iota(jnp.int32, (1, B), 1)
        oh = jnp.where(jnp.logical_and(mask, batch_ref[...] == gid), 1.0, 0.0)
        psum[...] += jax.lax.dot_general(
            oh, h, (((0,), (0,)), ((), ())),
            preferred_element_type=jnp.float32)
        pcnt[...] += jax.lax.dot_general(
            oh, jnp.ones((RB, 1), jnp.float32), (((0,), (0,)), ((), ())),
            preferred_element_type=jnp.float32)
        for g in range(B):
            colmask = oh[:, g:g + 1] > 0
            mg = jnp.max(jnp.where(colmask, h, -3e38), axis=0, keepdims=True)
            pmax[g:g + 1, :] = jnp.maximum(pmax[g:g + 1, :], mg)

    @pl.when(jnp.logical_and(p == 2, blk == 0))
    def _():
        cnt = pcnt[...]
        xmean = psum[...] / jnp.maximum(cnt, 1.0)
        xmax = jnp.where(cnt > 0, pmax[...], 0.0)
        gemb = jnp.concatenate([xmean, xmax], axis=1)
        gemb_ref[...] = gemb
        z = jnp.maximum(
            jnp.dot(gemb, F1_ref[...], preferred_element_type=jnp.float32)
            + fb1_ref[...], 0.0)
        z = jnp.maximum(
            jnp.dot(z, F2_ref[...], preferred_element_type=jnp.float32)
            + fb2_ref[...], 0.0)
        logits_ref[...] = jnp.dot(
            z, F3_ref[...], preferred_element_type=jnp.float32) + fb3_ref[...]


def _k3(acc3, u3, dinv, b3, g3, be3, batch2d, F1, fb1, F2, fb2, F3, fb3):
    full = lambda shape: pl.BlockSpec(shape, lambda p, b: tuple(
        0 for _ in shape))
    return pl.pallas_call(
        _k3_body,
        grid=(3, 16),
        in_specs=[
            pl.BlockSpec((NC, RB, CW), lambda p, b: (0, b, 0)),
            pl.BlockSpec((RB, HID), lambda p, b: (b, 0)),
            pl.BlockSpec((RB, 1), lambda p, b: (b, 0)),
            full((1, HID)),
            full((1, HID)),
            full((1, HID)),
            pl.BlockSpec((RB, 1), lambda p, b: (b, 0)),
            full((2 * HID, HID)),
            full((1, HID)),
            full((HID, HID // 2)),
            full((1, HID // 2)),
            full((HID // 2, 2)),
            full((1, 2)),
        ],
        out_specs=[
            full((B, 2)),
            full((B, 2 * HID)),
        ],
        out_shape=[
            jax.ShapeDtypeStruct((B, 2), jnp.float32),
            jax.ShapeDtypeStruct((B, 2 * HID), jnp.float32),
        ],
        scratch_shapes=[
            pltpu.VMEM((1, HID), jnp.float32),
            pltpu.VMEM((1, HID), jnp.float32),
            pltpu.VMEM((2, HID), jnp.float32),
            pltpu.VMEM((B, HID), jnp.float32),
            pltpu.VMEM((B, HID), jnp.float32),
            pltpu.VMEM((B, 1), jnp.float32),
        ],
    )(acc3, u3, dinv, b3, g3, be3, batch2d, F1, fb1, F2, fb2, F3, fb3)


def kernel(x, edge_index, batch, W1, b1, g1, be1, W2, b2, g2, be2, W3, b3, g3,
           be3, F1, fb1, F2, fb2, F3, fb3):
    src = edge_index[0]
    dst = edge_index[1]
    pad_e = EPAD - E
    srcp = jnp.concatenate([src, jnp.zeros((pad_e,), jnp.int32)])
    dstp = jnp.concatenate([dst, jnp.full((pad_e,), NPAD - 1, jnp.int32)])
    src2d = srcp.reshape(EPAD // 128, 128)
    dst2d = dstp.reshape(EPAD // 128, 128)
    x_pad = jnp.pad(x, ((0, NPAD - N), (0, 0)))
    batch2d = jnp.pad(batch, (0, NPAD - N),
                      constant_values=B - 1).reshape(NPAD, 1)

    degp = _sc_degree(dst2d)
    dinv, u1 = _k1(degp, x_pad, W1)
    acc1 = _sc_scatter(u1.reshape(NC * NPAD, CW), src2d, dst2d)
    u2a, u2b = _k2a(acc1, u1, dinv, b1.reshape(1, -1), g1.reshape(1, -1),
                    be1.reshape(1, -1), W2)
    acc2a = _sc_scatter(u2a.reshape(NC * NPAD, CW), src2d, dst2d)
    acc2b = _sc_scatter(u2b.reshape(NC * NPAD, CW), src2d, dst2d)
    (u3,) = _k2b(acc2a, acc2b, u2a, u2b, dinv, b2.reshape(1, -1),
                 g2.reshape(1, -1), be2.reshape(1, -1), W3)
    acc3 = _sc_scatter(u3.reshape(NC * NPAD, CW), src2d, dst2d)
    logits, gemb = _k3(acc3, u3, dinv, b3.reshape(1, -1), g3.reshape(1, -1),
                       be3.reshape(1, -1), batch2d, F1, fb1.reshape(1, -1),
                       F2, fb2.reshape(1, -1), F3, fb3.reshape(1, -1))
    return (logits, gemb)


# speedup vs baseline: 11.0191x; 1.0990x over previous
import functools

import jax
import jax.numpy as jnp
from jax import lax
from jax.experimental import pallas as pl
from jax.experimental.pallas import tpu as pltpu
from jax.experimental.pallas import tpu_sc as plsc

N = 50000
E = 800000
IN_DIM = 12
HID = 64
B = 64
EPS = 1e-5

NC = 2
NS = 16
L = 16

NPAD = 51200
EPAD = 819200
CW = HID // NC
ROWS_SUB = EPAD // NS // 128
CHUNK_ROWS = 20
NCHUNK = ROWS_SUB // CHUNK_ROWS
NSLOT = 4
ZROWS = NPAD // NS
RB = NPAD // 16

_mesh = plsc.VectorSubcoreMesh(core_axis_name="c", subcore_axis_name="s")


EPW = EPAD // (NC * NS)
EROWS_W = EPW // 128
HSLICE = NPAD // NS


@functools.partial(
    pl.kernel,
    out_type=jax.ShapeDtypeStruct((NC, NPAD), jnp.float32),
    mesh=_mesh,
    scratch_types=[
        pltpu.VMEM_SHARED((NPAD,), jnp.float32),
        pltpu.VMEM((HSLICE,), jnp.float32),
        pltpu.VMEM((EROWS_W, 128), jnp.int32),
        pltpu.VMEM((128,), jnp.float32),
        pltpu.SemaphoreType.DMA,
    ],
)
def _sc_degree(dst2d_hbm, out_hbm, hist, zbuf, buf, ones, sem):
    c = lax.axis_index("c")
    s = lax.axis_index("s")
    w = s * NC + c

    def zero_body(i, _):
        zbuf[pl.ds(i * L, L)] = jnp.zeros((L,), jnp.float32)
        return 0

    lax.fori_loop(0, HSLICE // L, zero_body, 0)
    for i in range(128 // L):
        ones[pl.ds(i * L, L)] = jnp.ones((L,), jnp.float32)
    pltpu.sync_copy(zbuf, hist.at[pl.ds(s * HSLICE, HSLICE)])
    pltpu.sync_copy(dst2d_hbm.at[pl.ds(w * EROWS_W, EROWS_W), :], buf)
    plsc.subcore_barrier()

    def add_body(jg, _):
        descs = [
            pltpu.async_copy(ones, hist.at[buf.at[jg * 10 + jj]], sem,
                             add=True)
            for jj in range(10)
        ]
        for d in descs:
            d.wait()
        return 0

    lax.fori_loop(0, EROWS_W // 10, add_body, 0)
    plsc.subcore_barrier()
    pltpu.sync_copy(hist.at[pl.ds(s * HSLICE, HSLICE)],
                    out_hbm.at[c, pl.ds(s * HSLICE, HSLICE)])


@functools.partial(
    pl.kernel,
    out_type=jax.ShapeDtypeStruct((NC, NPAD, CW), jnp.float32),
    mesh=_mesh,
    compiler_params=pltpu.CompilerParams(use_tc_tiling_on_sc=False),
    scratch_types=[
        pltpu.VMEM_SHARED((NPAD, CW), jnp.float32),
        pltpu.VMEM((NSLOT * 128, CW), jnp.float32),
        pltpu.VMEM((CHUNK_ROWS, 128), jnp.int32),
        pltpu.VMEM((CHUNK_ROWS, 128), jnp.int32),
        pltpu.SemaphoreType.DMA,
        pltpu.SemaphoreType.DMA,
        pltpu.SemaphoreType.DMA,
        pltpu.SemaphoreType.DMA,
        pltpu.SemaphoreType.DMA,
        pltpu.SemaphoreType.DMA,
        pltpu.SemaphoreType.DMA,
        pltpu.SemaphoreType.DMA,
    ],
)
def _sc_scatter(u2_hbm, src2d_hbm, dst2d_hbm, out_hbm, acc, rowbuf, srcbuf,
                dstbuf, g0, g1, g2, g3, a0, a1, a2, a3):
    c = lax.axis_index("c")
    s = lax.axis_index("s")
    gsems = (g0, g1, g2, g3)
    asems = (a0, a1, a2, a3)

    def zrow(i, _):
        for k in range(CW // L):
            rowbuf[i, pl.ds(k * L, L)] = jnp.zeros((L,), jnp.float32)
        return 0

    lax.fori_loop(0, 128, zrow, 0)

    zd = [
        pltpu.async_copy(rowbuf.at[pl.ds(0, 128), :],
                         acc.at[pl.ds(s * ZROWS + i * 128, 128), :], g0)
        for i in range(ZROWS // 128)
    ]
    for d in zd:
        d.wait()
    plsc.subcore_barrier()

    def chunk_body(b, _):
        row0 = s * ROWS_SUB + b * CHUNK_ROWS
        pltpu.sync_copy(src2d_hbm.at[pl.ds(row0, CHUNK_ROWS), :], srcbuf)
        pltpu.sync_copy(dst2d_hbm.at[pl.ds(row0, CHUNK_ROWS), :], dstbuf)
        for j in range(CHUNK_ROWS):
            for k in range(128 // L):
                v = srcbuf[j, pl.ds(k * L, L)]
                srcbuf[j, pl.ds(k * L, L)] = v * 2 + c

        gd = [None] * CHUNK_ROWS
        ad = [None] * CHUNK_ROWS
        for j in range(2):
            gd[j] = pltpu.async_copy(
                u2_hbm.at[srcbuf.at[j]],
                rowbuf.at[pl.ds(j * 128, 128), :], gsems[j])
        for j in range(CHUNK_ROWS):
            slot = j % NSLOT
            gd[j].wait()
            ad[j] = pltpu.async_copy(
                rowbuf.at[pl.ds(slot * 128, 128), :],
                acc.at[dstbuf.at[j]], asems[slot], add=True)
            nxt = j + 2
            if nxt < CHUNK_ROWS:
                prev = nxt - NSLOT
                if prev >= 0:
                    ad[prev].wait()
                gd[nxt] = pltpu.async_copy(
                    u2_hbm.at[srcbuf.at[nxt]],
                    rowbuf.at[pl.ds((nxt % NSLOT) * 128, 128), :],
                    gsems[nxt % NSLOT])
        for j in range(CHUNK_ROWS - NSLOT, CHUNK_ROWS):
            ad[j].wait()
        return 0

    lax.fori_loop(0, NCHUNK, chunk_body, 0)
    plsc.subcore_barrier()
    pltpu.sync_copy(acc.at[pl.ds(s * ZROWS, ZROWS), :],
                    out_hbm.at[c, pl.ds(s * ZROWS, ZROWS), :])


RB1 = RB


def _k1_body(degp_ref, x_ref, W1_ref, dinv_ref, u1_ref):
    pid = pl.program_id(0)
    deg = jnp.sum(degp_ref[...], axis=0) + 1.0
    rows = jax.lax.broadcasted_iota(jnp.int32, (RB1, 1), 0) + pid * RB1
    dinv = jnp.where(rows < N, jax.lax.rsqrt(deg)[:, None], 0.0)
    dinv_ref[...] = dinv
    h = jnp.dot(x_ref[...], W1_ref[...], preferred_element_type=jnp.float32)
    u1_ref[...] = h * dinv


def _k1(degp, x_pad, W1):
    return pl.pallas_call(
        _k1_body,
        grid=(16,),
        in_specs=[
            pl.BlockSpec((NC, RB1), lambda i: (0, i)),
            pl.BlockSpec((RB1, IN_DIM), lambda i: (i, 0)),
            pl.BlockSpec((IN_DIM, HID), lambda i: (0, 0)),
        ],
        out_specs=[
            pl.BlockSpec((RB1, 1), lambda i: (i, 0)),
            pl.BlockSpec((RB1, HID), lambda i: (i, 0)),
        ],
        out_shape=[
            jax.ShapeDtypeStruct((NPAD, 1), jnp.float32),
            jax.ShapeDtypeStruct((NPAD, HID), jnp.float32),
        ],
    )(degp, x_pad, W1)


def _k2a_body(acc_ref, u_ref, dinv_ref, b_ref, g_ref, be_ref, W_ref, ua_ref,
              ub_ref, sums, sumsq, mv):
    p = pl.program_id(0)
    blk = pl.program_id(1)
    dinv = dinv_ref[...]
    neigh = jnp.concatenate([acc_ref[0], acc_ref[1]], axis=1) + u_ref[...]
    agg = dinv * neigh + b_ref[...]
    rows = jax.lax.broadcasted_iota(jnp.int32, (RB, 1), 0) + blk * RB
    mask = rows < N

    @pl.when(jnp.logical_and(p == 0, blk == 0))
    def _():
        sums[...] = jnp.zeros_like(sums)
        sumsq[...] = jnp.zeros_like(sumsq)

    @pl.when(p == 0)
    def _():
        am = jnp.where(mask, agg, 0.0)
        sums[...] += jnp.sum(am, axis=0, keepdims=True)
        sumsq[...] += jnp.sum(jnp.where(mask, agg * agg, 0.0), axis=0,
                              keepdims=True)

    @pl.when(jnp.logical_and(p == 0, blk == 15))
    def _():
        m = sums[...] / N
        v = sumsq[...] / N - m * m
        mv[0:1, :] = m
        mv[1:2, :] = jax.lax.rsqrt(v + EPS)

    @pl.when(p == 1)
    def _():
        xn = (agg - mv[0:1, :]) * mv[1:2, :] * g_ref[...] + be_ref[...]
        h = jnp.maximum(xn, 0.0)
        u = jnp.dot(h, W_ref[...], preferred_element_type=jnp.float32) * dinv
        ua_ref[...] = u[:, :HID]
        ub_ref[...] = u[:, HID:]


def _k2a(acc1, u1, dinv, b1, g1, be1, W2):
    return pl.pallas_call(
        _k2a_body,
        grid=(2, 16),
        in_specs=[
            pl.BlockSpec((NC, RB, CW), lambda p, b: (0, b, 0)),
            pl.BlockSpec((RB, HID), lambda p, b: (b, 0)),
            pl.BlockSpec((RB, 1), lambda p, b: (b, 0)),
            pl.BlockSpec((1, HID), lambda p, b: (0, 0)),
            pl.BlockSpec((1, HID), lambda p, b: (0, 0)),
            pl.BlockSpec((1, HID), lambda p, b: (0, 0)),
            pl.BlockSpec((HID, 2 * HID), lambda p, b: (0, 0)),
        ],
        out_specs=[
            pl.BlockSpec((RB, HID), lambda p, b: (b, 0)),
            pl.BlockSpec((RB, HID), lambda p, b: (b, 0)),
        ],
        out_shape=[
            jax.ShapeDtypeStruct((NPAD, HID), jnp.float32),
            jax.ShapeDtypeStruct((NPAD, HID), jnp.float32),
        ],
        scratch_shapes=[
            pltpu.VMEM((1, HID), jnp.float32),
            pltpu.VMEM((1, HID), jnp.float32),
            pltpu.VMEM((2, HID), jnp.float32),
        ],
    )(acc1, u1, dinv, b1, g1, be1, W2)


def _k2b_body(acca_ref, accb_ref, ua_ref, ub_ref, dinv_ref, b_ref, g_ref,
              be_ref, W_ref, u_ref, sums, sumsq, mv):
    p = pl.program_id(0)
    blk = pl.program_id(1)
    dinv = dinv_ref[...]
    neigh = jnp.concatenate(
        [acca_ref[0], acca_ref[1], accb_ref[0], accb_ref[1]], axis=1)
    selfu = jnp.concatenate([ua_ref[...], ub_ref[...]], axis=1)
    agg = dinv * (neigh + selfu) + b_ref[...]
    rows = jax.lax.broadcasted_iota(jnp.int32, (RB, 1), 0) + blk * RB
    mask = rows < N

    @pl.when(jnp.logical_and(p == 0, blk == 0))
    def _():
        sums[...] = jnp.zeros_like(sums)
        sumsq[...] = jnp.zeros_like(sumsq)

    @pl.when(p == 0)
    def _():
        sums[...] += jnp.sum(jnp.where(mask, agg, 0.0), axis=0, keepdims=True)
        sumsq[...] += jnp.sum(jnp.where(mask, agg * agg, 0.0), axis=0,
                              keepdims=True)

    @pl.when(jnp.logical_and(p == 0, blk == 15))
    def _():
        m = sums[...] / N
        v = sumsq[...] / N - m * m
        mv[0:1, :] = m
        mv[1:2, :] = jax.lax.rsqrt(v + EPS)

    @pl.when(p == 1)
    def _():
        xn = (agg - mv[0:1, :]) * mv[1:2, :] * g_ref[...] + be_ref[...]
        h = jnp.maximum(xn, 0.0)
        u_ref[...] = jnp.dot(h, W_ref[...],
                             preferred_element_type=jnp.float32) * dinv


def _k2b(acca, accb, ua, ub, dinv, b2, g2, be2, W3):
    return pl.pallas_call(
        _k2b_body,
        grid=(2, 16),
        in_specs=[
            pl.BlockSpec((NC, RB, CW), lambda p, b: (0, b, 0)),
            pl.BlockSpec((NC, RB, CW), lambda p, b: (0, b, 0)),
            pl.BlockSpec((RB, HID), lambda p, b: (b, 0)),
            pl.BlockSpec((RB, HID), lambda p, b: (b, 0)),
            pl.BlockSpec((RB, 1), lambda p, b: (b, 0)),
            pl.BlockSpec((1, 2 * HID), lambda p, b: (0, 0)),
            pl.BlockSpec((1, 2 * HID), lambda p, b: (0, 0)),
            pl.BlockSpec((1, 2 * HID), lambda p, b: (0, 0)),
            pl.BlockSpec((2 * HID, HID), lambda p, b: (0, 0)),
        ],
        out_specs=[pl.BlockSpec((RB, HID), lambda p, b: (b, 0))],
        out_shape=[jax.ShapeDtypeStruct((NPAD, HID), jnp.float32)],
        scratch_shapes=[
            pltpu.VMEM((1, 2 * HID), jnp.float32),
            pltpu.VMEM((1, 2 * HID), jnp.float32),
            pltpu.VMEM((2, 2 * HID), jnp.float32),
        ],
    )(acca, accb, ua, ub, dinv, b2, g2, be2, W3)


def _k3_body(acc_ref, u_ref, dinv_ref, b_ref, g_ref, be_ref, batch_ref,
             F1_ref, fb1_ref, F2_ref, fb2_ref, F3_ref, fb3_ref, logits_ref,
             gemb_ref, sums, sumsq, mv, psum, pmax, pcnt):
    p = pl.program_id(0)
    blk = pl.program_id(1)
    neigh = jnp.concatenate([acc_ref[0], acc_ref[1]], axis=1) + u_ref[...]
    agg = dinv_ref[...] * neigh + b_ref[...]
    rows = jax.lax.broadcasted_iota(jnp.int32, (RB, 1), 0) + blk * RB
    mask = rows < N

    @pl.when(jnp.logical_and(p == 0, blk == 0))
    def _():
        sums[...] = jnp.zeros_like(sums)
        sumsq[...] = jnp.zeros_like(sumsq)

    @pl.when(p == 0)
    def _():
        sums[...] += jnp.sum(jnp.where(mask, agg, 0.0), axis=0, keepdims=True)
        sumsq[...] += jnp.sum(jnp.where(mask, agg * agg, 0.0), axis=0,
                              keepdims=True)

    @pl.when(jnp.logical_and(p == 0, blk == 15))
    def _():
        m = sums[...] / N
        v = sumsq[...] / N - m * m
        mv[0:1, :] = m
        mv[1:2, :] = jax.lax.rsqrt(v + EPS)

    @pl.when(jnp.logical_and(p == 1, blk == 0))
    def _():
        psum[...] = jnp.zeros_like(psum)
        pmax[...] = jnp.full_like(pmax, -3e38)
        pcnt[...] = jnp.zeros_like(pcnt)

    @pl.when(p == 1)
    def _():
        xn = (agg - mv[0:1, :]) * mv[1:2, :] * g_ref[...] + be_ref[...]
        h = jnp.maximum(xn, 0.0)
        bvals = batch_ref[...]
        gid = jax.lax.broadcasted_iota(jnp.int32, (1, B), 1)
        oh = jnp.where(jnp.logical_and(mask, bvals == gid), 1.0, 0.0)
        psum[...] += jax.lax.dot_general(
            oh, h, (((0,), (0,)), ((), ())),
            preferred_element_type=jnp.float32)
        pcnt[...] += jax.lax.dot_general(
            oh, jnp.ones((RB, 1), jnp.float32), (((0,), (0,)), ((), ())),
            preferred_element_type=jnp.float32)
        gmin = jnp.min(jnp.where(mask, bvals, B - 1))
        gmax = jnp.max(jnp.where(mask, bvals, 0))

        def gbody(g, _):
            sel = jnp.logical_and(mask, bvals == g)
            mg = jnp.max(jnp.where(sel, h, -3e38), axis=0, keepdims=True)
            pmax[pl.ds(g, 1), :] = jnp.maximum(pmax[pl.ds(g, 1), :], mg)
            return 0

        lax.fori_loop(gmin, gmax + 1, gbody, 0)

    @pl.when(jnp.logical_and(p == 2, blk == 0))
    def _():
        cnt = pcnt[...]
        xmean = psum[...] / jnp.maximum(cnt, 1.0)
        xmax = jnp.where(cnt > 0, pmax[...], 0.0)
        gemb = jnp.concatenate([xmean, xmax], axis=1)
        gemb_ref[...] = gemb
        z = jnp.maximum(
            jnp.dot(gemb, F1_ref[...], preferred_element_type=jnp.float32)
            + fb1_ref[...], 0.0)
        z = jnp.maximum(
            jnp.dot(z, F2_ref[...], preferred_element_type=jnp.float32)
            + fb2_ref[...], 0.0)
        logits_ref[...] = jnp.dot(
            z, F3_ref[...], preferred_element_type=jnp.float32) + fb3_ref[...]


def _k3(acc3, u3, dinv, b3, g3, be3, batch2d, F1, fb1, F2, fb2, F3, fb3):
    full = lambda shape: pl.BlockSpec(shape, lambda p, b: tuple(
        0 for _ in shape))
    return pl.pallas_call(
        _k3_body,
        grid=(3, 16),
        in_specs=[
            pl.BlockSpec((NC, RB, CW), lambda p, b: (0, b, 0)),
            pl.BlockSpec((RB, HID), lambda p, b: (b, 0)),
            pl.BlockSpec((RB, 1), lambda p, b: (b, 0)),
            full((1, HID)),
            full((1, HID)),
            full((1, HID)),
            pl.BlockSpec((RB, 1), lambda p, b: (b, 0)),
            full((2 * HID, HID)),
            full((1, HID)),
            full((HID, HID // 2)),
            full((1, HID // 2)),
            full((HID // 2, 2)),
            full((1, 2)),
        ],
        out_specs=[
            full((B, 2)),
            full((B, 2 * HID)),
        ],
        out_shape=[
            jax.ShapeDtypeStruct((B, 2), jnp.float32),
            jax.ShapeDtypeStruct((B, 2 * HID), jnp.float32),
        ],
        scratch_shapes=[
            pltpu.VMEM((1, HID), jnp.float32),
            pltpu.VMEM((1, HID), jnp.float32),
            pltpu.VMEM((2, HID), jnp.float32),
            pltpu.VMEM((B, HID), jnp.float32),
            pltpu.VMEM((B, HID), jnp.float32),
            pltpu.VMEM((B, 1), jnp.float32),
        ],
    )(acc3, u3, dinv, b3, g3, be3, batch2d, F1, fb1, F2, fb2, F3, fb3)


def kernel(x, edge_index, batch, W1, b1, g1, be1, W2, b2, g2, be2, W3, b3, g3,
           be3, F1, fb1, F2, fb2, F3, fb3):
    src = edge_index[0]
    dst = edge_index[1]
    pad_e = EPAD - E
    srcp = jnp.concatenate([src, jnp.zeros((pad_e,), jnp.int32)])
    dstp = jnp.concatenate([dst, jnp.full((pad_e,), NPAD - 1, jnp.int32)])
    src2d = srcp.reshape(EPAD // 128, 128)
    dst2d = dstp.reshape(EPAD // 128, 128)
    x_pad = jnp.pad(x, ((0, NPAD - N), (0, 0)))
    batch2d = jnp.pad(batch, (0, NPAD - N),
                      constant_values=B - 1).reshape(NPAD, 1)

    degp = _sc_degree(dst2d)
    dinv, u1 = _k1(degp, x_pad, W1)
    acc1 = _sc_scatter(u1.reshape(NC * NPAD, CW), src2d, dst2d)
    u2a, u2b = _k2a(acc1, u1, dinv, b1.reshape(1, -1), g1.reshape(1, -1),
                    be1.reshape(1, -1), W2)
    acc2a = _sc_scatter(u2a.reshape(NC * NPAD, CW), src2d, dst2d)
    acc2b = _sc_scatter(u2b.reshape(NC * NPAD, CW), src2d, dst2d)
    (u3,) = _k2b(acc2a, acc2b, u2a, u2b, dinv, b2.reshape(1, -1),
                 g2.reshape(1, -1), be2.reshape(1, -1), W3)
    acc3 = _sc_scatter(u3.reshape(NC * NPAD, CW), src2d, dst2d)
    logits, gemb = _k3(acc3, u3, dinv, b3.reshape(1, -1), g3.reshape(1, -1),
                       be3.reshape(1, -1), batch2d, F1, fb1.reshape(1, -1),
                       F2, fb2.reshape(1, -1), F3, fb3.reshape(1, -1))
    return (logits, gemb)


# speedup vs baseline: 11.1927x; 1.0157x over previous
import functools

import jax
import jax.numpy as jnp
from jax import lax
from jax.experimental import pallas as pl
from jax.experimental.pallas import tpu as pltpu
from jax.experimental.pallas import tpu_sc as plsc

N = 50000
E = 800000
IN_DIM = 12
HID = 64
B = 64
EPS = 1e-5

NC = 2
NS = 16
L = 16

NPAD = 51200
EPAD = 819200
CW = HID // NC
ROWS_SUB = EPAD // NS // 128
CHUNK_ROWS = 20
NCHUNK = ROWS_SUB // CHUNK_ROWS
NSLOT = 4
ZROWS = NPAD // NS
RB = NPAD // 16

_mesh = plsc.VectorSubcoreMesh(core_axis_name="c", subcore_axis_name="s")


EPW = EPAD // (NC * NS)
EROWS_W = EPW // 128
HSLICE = NPAD // NS


@functools.partial(
    pl.kernel,
    out_type=jax.ShapeDtypeStruct((NC, NPAD), jnp.float32),
    mesh=_mesh,
    scratch_types=[
        pltpu.VMEM_SHARED((NPAD,), jnp.float32),
        pltpu.VMEM((HSLICE,), jnp.float32),
        pltpu.VMEM((EROWS_W, 128), jnp.int32),
        pltpu.VMEM((128,), jnp.float32),
        pltpu.SemaphoreType.DMA,
    ],
)
def _sc_degree(dst2d_hbm, out_hbm, hist, zbuf, buf, ones, sem):
    c = lax.axis_index("c")
    s = lax.axis_index("s")
    w = s * NC + c

    def zero_body(i, _):
        zbuf[pl.ds(i * L, L)] = jnp.zeros((L,), jnp.float32)
        return 0

    lax.fori_loop(0, HSLICE // L, zero_body, 0)
    for i in range(128 // L):
        ones[pl.ds(i * L, L)] = jnp.ones((L,), jnp.float32)
    pltpu.sync_copy(zbuf, hist.at[pl.ds(s * HSLICE, HSLICE)])
    pltpu.sync_copy(dst2d_hbm.at[pl.ds(w * EROWS_W, EROWS_W), :], buf)
    plsc.subcore_barrier()

    def add_body(jg, _):
        descs = [
            pltpu.async_copy(ones, hist.at[buf.at[jg * 10 + jj]], sem,
                             add=True)
            for jj in range(10)
        ]
        for d in descs:
            d.wait()
        return 0

    lax.fori_loop(0, EROWS_W // 10, add_body, 0)
    plsc.subcore_barrier()
    pltpu.sync_copy(hist.at[pl.ds(s * HSLICE, HSLICE)],
                    out_hbm.at[c, pl.ds(s * HSLICE, HSLICE)])


@functools.partial(
    pl.kernel,
    out_type=jax.ShapeDtypeStruct((NC, NPAD, CW), jnp.float32),
    mesh=_mesh,
    compiler_params=pltpu.CompilerParams(use_tc_tiling_on_sc=False),
    scratch_types=[
        pltpu.VMEM_SHARED((NPAD, CW), jnp.float32),
        pltpu.VMEM((NSLOT * 128, CW), jnp.float32),
        pltpu.VMEM((CHUNK_ROWS, 128), jnp.int32),
        pltpu.VMEM((CHUNK_ROWS, 128), jnp.int32),
        pltpu.SemaphoreType.DMA,
        pltpu.SemaphoreType.DMA,
        pltpu.SemaphoreType.DMA,
        pltpu.SemaphoreType.DMA,
        pltpu.SemaphoreType.DMA,
        pltpu.SemaphoreType.DMA,
        pltpu.SemaphoreType.DMA,
        pltpu.SemaphoreType.DMA,
    ],
)
def _sc_scatter(u2_hbm, src2d_hbm, dst2d_hbm, out_hbm, acc, rowbuf, srcbuf,
                dstbuf, g0, g1, g2, g3, a0, a1, a2, a3):
    c = lax.axis_index("c")
    s = lax.axis_index("s")
    gsems = (g0, g1, g2, g3)
    asems = (a0, a1, a2, a3)

    def zrow(i, _):
        for k in range(CW // L):
            rowbuf[i, pl.ds(k * L, L)] = jnp.zeros((L,), jnp.float32)
        return 0

    lax.fori_loop(0, 128, zrow, 0)

    zd = [
        pltpu.async_copy(rowbuf.at[pl.ds(0, 128), :],
                         acc.at[pl.ds(s * ZROWS + i * 128, 128), :], g0)
        for i in range(ZROWS // 128)
    ]
    for d in zd:
        d.wait()
    plsc.subcore_barrier()

    def chunk_body(b, _):
        row0 = s * ROWS_SUB + b * CHUNK_ROWS
        pltpu.sync_copy(src2d_hbm.at[pl.ds(row0, CHUNK_ROWS), :], srcbuf)
        pltpu.sync_copy(dst2d_hbm.at[pl.ds(row0, CHUNK_ROWS), :], dstbuf)
        for j in range(CHUNK_ROWS):
            for k in range(128 // L):
                v = srcbuf[j, pl.ds(k * L, L)]
                srcbuf[j, pl.ds(k * L, L)] = v * 2 + c

        gd = [None] * CHUNK_ROWS
        ad = [None] * CHUNK_ROWS
        for j in range(2):
            gd[j] = pltpu.async_copy(
                u2_hbm.at[srcbuf.at[j]],
                rowbuf.at[pl.ds(j * 128, 128), :], gsems[j])
        for j in range(CHUNK_ROWS):
            slot = j % NSLOT
            gd[j].wait()
            ad[j] = pltpu.async_copy(
                rowbuf.at[pl.ds(slot * 128, 128), :],
                acc.at[dstbuf.at[j]], asems[slot], add=True)
            nxt = j + 2
            if nxt < CHUNK_ROWS:
                prev = nxt - NSLOT
                if prev >= 0:
                    ad[prev].wait()
                gd[nxt] = pltpu.async_copy(
                    u2_hbm.at[srcbuf.at[nxt]],
                    rowbuf.at[pl.ds((nxt % NSLOT) * 128, 128), :],
                    gsems[nxt % NSLOT])
        for j in range(CHUNK_ROWS - NSLOT, CHUNK_ROWS):
            ad[j].wait()
        return 0

    lax.fori_loop(0, NCHUNK, chunk_body, 0)
    plsc.subcore_barrier()
    pltpu.sync_copy(acc.at[pl.ds(s * ZROWS, ZROWS), :],
                    out_hbm.at[c, pl.ds(s * ZROWS, ZROWS), :])


@functools.partial(
    pl.kernel,
    out_type=jax.ShapeDtypeStruct((2, NC, NPAD, CW), jnp.float32),
    mesh=_mesh,
    compiler_params=pltpu.CompilerParams(use_tc_tiling_on_sc=False),
    scratch_types=[
        pltpu.VMEM_SHARED((NPAD, CW), jnp.float32),
        pltpu.VMEM((NSLOT * 128, CW), jnp.float32),
        pltpu.VMEM((CHUNK_ROWS, 128), jnp.int32),
        pltpu.VMEM((CHUNK_ROWS, 128), jnp.int32),
        pltpu.SemaphoreType.DMA,
        pltpu.SemaphoreType.DMA,
        pltpu.SemaphoreType.DMA,
        pltpu.SemaphoreType.DMA,
        pltpu.SemaphoreType.DMA,
        pltpu.SemaphoreType.DMA,
        pltpu.SemaphoreType.DMA,
        pltpu.SemaphoreType.DMA,
    ],
)
def _sc_scatter2(u4_hbm, src2d_hbm, dst2d_hbm, out_hbm, acc, rowbuf, srcbuf,
                 dstbuf, g0, g1, g2, g3, a0, a1, a2, a3):
    c = lax.axis_index("c")
    s = lax.axis_index("s")
    gsems = (g0, g1, g2, g3)
    asems = (a0, a1, a2, a3)

    def zrow(i, _):
        for k in range(CW // L):
            rowbuf[i, pl.ds(k * L, L)] = jnp.zeros((L,), jnp.float32)
        return 0

    for t in range(2):
        lax.fori_loop(0, 128, zrow, 0)
        zd = [
            pltpu.async_copy(rowbuf.at[pl.ds(0, 128), :],
                             acc.at[pl.ds(s * ZROWS + i * 128, 128), :], g0)
            for i in range(ZROWS // 128)
        ]
        for d in zd:
            d.wait()
        plsc.subcore_barrier()

        def chunk_body(b, _):
            row0 = s * ROWS_SUB + b * CHUNK_ROWS
            pltpu.sync_copy(src2d_hbm.at[pl.ds(row0, CHUNK_ROWS), :], srcbuf)
            pltpu.sync_copy(dst2d_hbm.at[pl.ds(row0, CHUNK_ROWS), :], dstbuf)
            for j in range(CHUNK_ROWS):
                for k in range(128 // L):
                    v = srcbuf[j, pl.ds(k * L, L)]
                    srcbuf[j, pl.ds(k * L, L)] = v * 4 + (2 * t) + c

            gd = [None] * CHUNK_ROWS
            ad = [None] * CHUNK_ROWS
            for j in range(2):
                gd[j] = pltpu.async_copy(
                    u4_hbm.at[srcbuf.at[j]],
                    rowbuf.at[pl.ds(j * 128, 128), :], gsems[j])
            for j in range(CHUNK_ROWS):
                slot = j % NSLOT
                gd[j].wait()
                ad[j] = pltpu.async_copy(
                    rowbuf.at[pl.ds(slot * 128, 128), :],
                    acc.at[dstbuf.at[j]], asems[slot], add=True)
                nxt = j + 2
                if nxt < CHUNK_ROWS:
                    prev = nxt - NSLOT
                    if prev >= 0:
                        ad[prev].wait()
                    gd[nxt] = pltpu.async_copy(
                        u4_hbm.at[srcbuf.at[nxt]],
                        rowbuf.at[pl.ds((nxt % NSLOT) * 128, 128), :],
                        gsems[nxt % NSLOT])
            for j in range(CHUNK_ROWS - NSLOT, CHUNK_ROWS):
                ad[j].wait()
            return 0

        lax.fori_loop(0, NCHUNK, chunk_body, 0)
        plsc.subcore_barrier()
        pltpu.sync_copy(acc.at[pl.ds(s * ZROWS, ZROWS), :],
                        out_hbm.at[t, c, pl.ds(s * ZROWS, ZROWS), :])


RB1 = RB


def _k1_body(degp_ref, x_ref, W1_ref, dinv_ref, u1_ref):
    pid = pl.program_id(0)
    deg = jnp.sum(degp_ref[...], axis=0) + 1.0
    rows = jax.lax.broadcasted_iota(jnp.int32, (RB1, 1), 0) + pid * RB1
    dinv = jnp.where(rows < N, jax.lax.rsqrt(deg)[:, None], 0.0)
    dinv_ref[...] = dinv
    h = jnp.dot(x_ref[...], W1_ref[...], preferred_element_type=jnp.float32)
    u1_ref[...] = h * dinv


def _k1(degp, x_pad, W1):
    return pl.pallas_call(
        _k1_body,
        grid=(16,),
        in_specs=[
            pl.BlockSpec((NC, RB1), lambda i: (0, i)),
            pl.BlockSpec((RB1, IN_DIM), lambda i: (i, 0)),
            pl.BlockSpec((IN_DIM, HID), lambda i: (0, 0)),
        ],
        out_specs=[
            pl.BlockSpec((RB1, 1), lambda i: (i, 0)),
            pl.BlockSpec((RB1, HID), lambda i: (i, 0)),
        ],
        out_shape=[
            jax.ShapeDtypeStruct((NPAD, 1), jnp.float32),
            jax.ShapeDtypeStruct((NPAD, HID), jnp.float32),
        ],
    )(degp, x_pad, W1)


def _k2a_body(acc_ref, u_ref, dinv_ref, b_ref, g_ref, be_ref, W_ref, u2_ref,
              sums, sumsq, mv):
    p = pl.program_id(0)
    blk = pl.program_id(1)
    dinv = dinv_ref[...]
    neigh = jnp.concatenate([acc_ref[0], acc_ref[1]], axis=1) + u_ref[...]
    agg = dinv * neigh + b_ref[...]
    rows = jax.lax.broadcasted_iota(jnp.int32, (RB, 1), 0) + blk * RB
    mask = rows < N

    @pl.when(jnp.logical_and(p == 0, blk == 0))
    def _():
        sums[...] = jnp.zeros_like(sums)
        sumsq[...] = jnp.zeros_like(sumsq)

    @pl.when(p == 0)
    def _():
        am = jnp.where(mask, agg, 0.0)
        sums[...] += jnp.sum(am, axis=0, keepdims=True)
        sumsq[...] += jnp.sum(jnp.where(mask, agg * agg, 0.0), axis=0,
                              keepdims=True)

    @pl.when(jnp.logical_and(p == 0, blk == 15))
    def _():
        m = sums[...] / N
        v = sumsq[...] / N - m * m
        mv[0:1, :] = m
        mv[1:2, :] = jax.lax.rsqrt(v + EPS)

    @pl.when(p == 1)
    def _():
        xn = (agg - mv[0:1, :]) * mv[1:2, :] * g_ref[...] + be_ref[...]
        h = jnp.maximum(xn, 0.0)
        u2_ref[...] = jnp.dot(h, W_ref[...],
                              preferred_element_type=jnp.float32) * dinv


def _k2a(acc1, u1, dinv, b1, g1, be1, W2):
    return pl.pallas_call(
        _k2a_body,
        grid=(2, 16),
        in_specs=[
            pl.BlockSpec((NC, RB, CW), lambda p, b: (0, b, 0)),
            pl.BlockSpec((RB, HID), lambda p, b: (b, 0)),
            pl.BlockSpec((RB, 1), lambda p, b: (b, 0)),
            pl.BlockSpec((1, HID), lambda p, b: (0, 0)),
            pl.BlockSpec((1, HID), lambda p, b: (0, 0)),
            pl.BlockSpec((1, HID), lambda p, b: (0, 0)),
            pl.BlockSpec((HID, 2 * HID), lambda p, b: (0, 0)),
        ],
        out_specs=[
            pl.BlockSpec((RB, 2 * HID), lambda p, b: (b, 0)),
        ],
        out_shape=[
            jax.ShapeDtypeStruct((NPAD, 2 * HID), jnp.float32),
        ],
        scratch_shapes=[
            pltpu.VMEM((1, HID), jnp.float32),
            pltpu.VMEM((1, HID), jnp.float32),
            pltpu.VMEM((2, HID), jnp.float32),
        ],
    )(acc1, u1, dinv, b1, g1, be1, W2)


def _k2b_body(acc_ref, u2_ref, dinv_ref, b_ref, g_ref, be_ref, W_ref, u_ref,
              sums, sumsq, mv):
    p = pl.program_id(0)
    blk = pl.program_id(1)
    dinv = dinv_ref[...]
    neigh = jnp.concatenate(
        [acc_ref[0, 0], acc_ref[0, 1], acc_ref[1, 0], acc_ref[1, 1]], axis=1)
    agg = dinv * (neigh + u2_ref[...]) + b_ref[...]
    rows = jax.lax.broadcasted_iota(jnp.int32, (RB, 1), 0) + blk * RB
    mask = rows < N

    @pl.when(jnp.logical_and(p == 0, blk == 0))
    def _():
        sums[...] = jnp.zeros_like(sums)
        sumsq[...] = jnp.zeros_like(sumsq)

    @pl.when(p == 0)
    def _():
        sums[...] += jnp.sum(jnp.where(mask, agg, 0.0), axis=0, keepdims=True)
        sumsq[...] += jnp.sum(jnp.where(mask, agg * agg, 0.0), axis=0,
                              keepdims=True)

    @pl.when(jnp.logical_and(p == 0, blk == 15))
    def _():
        m = sums[...] / N
        v = sumsq[...] / N - m * m
        mv[0:1, :] = m
        mv[1:2, :] = jax.lax.rsqrt(v + EPS)

    @pl.when(p == 1)
    def _():
        xn = (agg - mv[0:1, :]) * mv[1:2, :] * g_ref[...] + be_ref[...]
        h = jnp.maximum(xn, 0.0)
        u_ref[...] = jnp.dot(h, W_ref[...],
                             preferred_element_type=jnp.float32) * dinv


def _k2b(acc2, u2, dinv, b2, g2, be2, W3):
    return pl.pallas_call(
        _k2b_body,
        grid=(2, 16),
        in_specs=[
            pl.BlockSpec((2, NC, RB, CW), lambda p, b: (0, 0, b, 0)),
            pl.BlockSpec((RB, 2 * HID), lambda p, b: (b, 0)),
            pl.BlockSpec((RB, 1), lambda p, b: (b, 0)),
            pl.BlockSpec((1, 2 * HID), lambda p, b: (0, 0)),
            pl.BlockSpec((1, 2 * HID), lambda p, b: (0, 0)),
            pl.BlockSpec((1, 2 * HID), lambda p, b: (0, 0)),
            pl.BlockSpec((2 * HID, HID), lambda p, b: (0, 0)),
        ],
        out_specs=[pl.BlockSpec((RB, HID), lambda p, b: (b, 0))],
        out_shape=[jax.ShapeDtypeStruct((NPAD, HID), jnp.float32)],
        scratch_shapes=[
            pltpu.VMEM((1, 2 * HID), jnp.float32),
            pltpu.VMEM((1, 2 * HID), jnp.float32),
            pltpu.VMEM((2, 2 * HID), jnp.float32),
        ],
    )(acc2, u2, dinv, b2, g2, be2, W3)


def _k3_body(acc_ref, u_ref, dinv_ref, b_ref, g_ref, be_ref, batch_ref,
             F1_ref, fb1_ref, F2_ref, fb2_ref, F3_ref, fb3_ref, logits_ref,
             gemb_ref, sums, sumsq, mv, psum, pmax, pcnt):
    p = pl.program_id(0)
    blk = pl.program_id(1)
    neigh = jnp.concatenate([acc_ref[0], acc_ref[1]], axis=1) + u_ref[...]
    agg = dinv_ref[...] * neigh + b_ref[...]
    rows = jax.lax.broadcasted_iota(jnp.int32, (RB, 1), 0) + blk * RB
    mask = rows < N

    @pl.when(jnp.logical_and(p == 0, blk == 0))
    def _():
        sums[...] = jnp.zeros_like(sums)
        sumsq[...] = jnp.zeros_like(sumsq)

    @pl.when(p == 0)
    def _():
        sums[...] += jnp.sum(jnp.where(mask, agg, 0.0), axis=0, keepdims=True)
        sumsq[...] += jnp.sum(jnp.where(mask, agg * agg, 0.0), axis=0,
                              keepdims=True)

    @pl.when(jnp.logical_and(p == 0, blk == 15))
    def _():
        m = sums[...] / N
        v = sumsq[...] / N - m * m
        mv[0:1, :] = m
        mv[1:2, :] = jax.lax.rsqrt(v + EPS)

    @pl.when(jnp.logical_and(p == 1, blk == 0))
    def _():
        psum[...] = jnp.zeros_like(psum)
        pmax[...] = jnp.full_like(pmax, -3e38)
        pcnt[...] = jnp.zeros_like(pcnt)

    @pl.when(p == 1)
    def _():
        xn = (agg - mv[0:1, :]) * mv[1:2, :] * g_ref[...] + be_ref[...]
        h = jnp.maximum(xn, 0.0)
        bvals = batch_ref[...]
        gid = jax.lax.broadcasted_iota(jnp.int32, (1, B), 1)
        oh = jnp.where(jnp.logical_and(mask, bvals == gid), 1.0, 0.0)
        psum[...] += jax.lax.dot_general(
            oh, h, (((0,), (0,)), ((), ())),
            preferred_element_type=jnp.float32)
        pcnt[...] += jax.lax.dot_general(
            oh, jnp.ones((RB, 1), jnp.float32), (((0,), (0,)), ((), ())),
            preferred_element_type=jnp.float32)
        gmin = jnp.min(jnp.where(mask, bvals, B - 1))
        gmax = jnp.max(jnp.where(mask, bvals, 0))

        def gbody(g, _):
            sel = jnp.logical_and(mask, bvals == g)
            mg = jnp.max(jnp.where(sel, h, -3e38), axis=0, keepdims=True)
            pmax[pl.ds(g, 1), :] = jnp.maximum(pmax[pl.ds(g, 1), :], mg)
            return 0

        lax.fori_loop(gmin, gmax + 1, gbody, 0)

    @pl.when(jnp.logical_and(p == 2, blk == 0))
    def _():
        cnt = pcnt[...]
        xmean = psum[...] / jnp.maximum(cnt, 1.0)
        xmax = jnp.where(cnt > 0, pmax[...], 0.0)
        gemb = jnp.concatenate([xmean, xmax], axis=1)
        gemb_ref[...] = gemb
        z = jnp.maximum(
            jnp.dot(gemb, F1_ref[...], preferred_element_type=jnp.float32)
            + fb1_ref[...], 0.0)
        z = jnp.maximum(
            jnp.dot(z, F2_ref[...], preferred_element_type=jnp.float32)
            + fb2_ref[...], 0.0)
        logits_ref[...] = jnp.dot(
            z, F3_ref[...], preferred_element_type=jnp.float32) + fb3_ref[...]


def _k3(acc3, u3, dinv, b3, g3, be3, batch2d, F1, fb1, F2, fb2, F3, fb3):
    full = lambda shape: pl.BlockSpec(shape, lambda p, b: tuple(
        0 for _ in shape))
    return pl.pallas_call(
        _k3_body,
        grid=(3, 16),
        in_specs=[
            pl.BlockSpec((NC, RB, CW), lambda p, b: (0, b, 0)),
            pl.BlockSpec((RB, HID), lambda p, b: (b, 0)),
            pl.BlockSpec((RB, 1), lambda p, b: (b, 0)),
            full((1, HID)),
            full((1, HID)),
            full((1, HID)),
            pl.BlockSpec((RB, 1), lambda p, b: (b, 0)),
            full((2 * HID, HID)),
            full((1, HID)),
            full((HID, HID // 2)),
            full((1, HID // 2)),
            full((HID // 2, 2)),
            full((1, 2)),
        ],
        out_specs=[
            full((B, 2)),
            full((B, 2 * HID)),
        ],
        out_shape=[
            jax.ShapeDtypeStruct((B, 2), jnp.float32),
            jax.ShapeDtypeStruct((B, 2 * HID), jnp.float32),
        ],
        scratch_shapes=[
            pltpu.VMEM((1, HID), jnp.float32),
            pltpu.VMEM((1, HID), jnp.float32),
            pltpu.VMEM((2, HID), jnp.float32),
            pltpu.VMEM((B, HID), jnp.float32),
            pltpu.VMEM((B, HID), jnp.float32),
            pltpu.VMEM((B, 1), jnp.float32),
        ],
    )(acc3, u3, dinv, b3, g3, be3, batch2d, F1, fb1, F2, fb2, F3, fb3)


def kernel(x, edge_index, batch, W1, b1, g1, be1, W2, b2, g2, be2, W3, b3, g3,
           be3, F1, fb1, F2, fb2, F3, fb3):
    src = edge_index[0]
    dst = edge_index[1]
    pad_e = EPAD - E
    srcp = jnp.concatenate([src, jnp.zeros((pad_e,), jnp.int32)])
    dstp = jnp.concatenate([dst, jnp.full((pad_e,), NPAD - 1, jnp.int32)])
    src2d = srcp.reshape(EPAD // 128, 128)
    dst2d = dstp.reshape(EPAD // 128, 128)
    x_pad = jnp.pad(x, ((0, NPAD - N), (0, 0)))
    batch2d = jnp.pad(batch, (0, NPAD - N),
                      constant_values=B - 1).reshape(NPAD, 1)

    degp = _sc_degree(dst2d)
    dinv, u1 = _k1(degp, x_pad, W1)
    acc1 = _sc_scatter(u1.reshape(NC * NPAD, CW), src2d, dst2d)
    (u2,) = _k2a(acc1, u1, dinv, b1.reshape(1, -1), g1.reshape(1, -1),
                 be1.reshape(1, -1), W2)
    acc2 = _sc_scatter2(u2.reshape(4 * NPAD, CW), src2d, dst2d)
    (u3,) = _k2b(acc2, u2, dinv, b2.reshape(1, -1),
                 g2.reshape(1, -1), be2.reshape(1, -1), W3)
    acc3 = _sc_scatter(u3.reshape(NC * NPAD, CW), src2d, dst2d)
    logits, gemb = _k3(acc3, u3, dinv, b3.reshape(1, -1), g3.reshape(1, -1),
                       be3.reshape(1, -1), batch2d, F1, fb1.reshape(1, -1),
                       F2, fb2.reshape(1, -1), F3, fb3.reshape(1, -1))
    return (logits, gemb)
